# baseline (speedup 1.0000x reference)
import functools

import jax
import jax.numpy as jnp
from jax import lax
from jax.experimental import pallas as pl
from jax.experimental.pallas import tpu as pltpu
from jax.experimental.pallas import tpu_sc as plsc

NC = 2
NS = 16
NW = NC * NS
K = 128
HH = 64
BLK = 256
NBUF = 2
IB = 40

F32 = jnp.float32


def _deg_body(dst_hbm, out_hbm, didx_all, ones_v, zb, acc, *, nchunks, npad):
  cid = lax.axis_index("c")
  sid = lax.axis_index("s")
  wid = cid * NS + sid
  rpt = npad // NS
  for c in range(8):
    zb[pl.ds(c * 16, 16)] = jnp.zeros((16,), F32)
    ones_v[pl.ds(c * 16, 16)] = jnp.full((16,), 1.0, F32)
  r0 = sid * rpt
  for t in range(rpt // K):
    pltpu.sync_copy(zb, acc.at[pl.ds(r0 + t * K, K)])
  pltpu.sync_copy(dst_hbm.at[wid], didx_all)
  plsc.subcore_barrier()

  def body(j, carry):
    pltpu.sync_copy(ones_v, acc.at[didx_all.at[j]], add=True)
    return carry

  lax.fori_loop(0, nchunks, body, 0)
  plsc.subcore_barrier()
  for t in range(rpt // K):
    pltpu.sync_copy(acc.at[pl.ds(r0 + t * K, K)],
                    out_hbm.at[cid, pl.ds(r0 + t * K, K)])


def _agg_body(zs_hbm, src_hbm, dst_hbm, out_hbm, sidx_blk, didx_blk,
              rows, zb, acc, *sems, nchunks, npad):
  cid = lax.axis_index("c")
  sid = lax.axis_index("s")
  rpt = npad // NS
  for i in range(16):
    for c in range(8):
      zb[i, pl.ds(c * 16, 16)] = jnp.zeros((16,), F32)
  r0 = sid * rpt
  for t in range(rpt // 16):
    pltpu.sync_copy(zb, acc.at[pl.ds(r0 + t * 16, 16)])
  wid = cid * NS + sid
  plsc.subcore_barrier()

  def outer(ib, carry):
    pltpu.sync_copy(src_hbm.at[wid, pl.ds(ib * IB, IB)], sidx_blk)
    pltpu.sync_copy(dst_hbm.at[wid, pl.ds(ib * IB, IB)], didx_blk)

    def body(i, carry2):
      handles = []
      for b in range(NBUF):
        c = i * NBUF + b
        handles.append(
            pltpu.async_copy(zs_hbm.at[sidx_blk.at[c]], rows.at[b], sems[b]))
      for b in range(NBUF):
        c = i * NBUF + b
        handles[b].wait()
        pltpu.sync_copy(rows.at[b], acc.at[didx_blk.at[c]], add=True)
      return carry2

    lax.fori_loop(0, IB // NBUF, body, 0)
    return carry

  lax.fori_loop(0, nchunks // IB, outer, 0)
  plsc.subcore_barrier()
  for t in range(rpt // K):
    pltpu.sync_copy(acc.at[pl.ds(r0 + t * K, K)],
                    out_hbm.at[cid, pl.ds(r0 + t * K, K)])


def _sc_deg(dst3, npad):
  nchunks = dst3.shape[1]
  mesh = plsc.VectorSubcoreMesh(core_axis_name="c", subcore_axis_name="s")
  fn = pl.kernel(
      functools.partial(_deg_body, nchunks=nchunks, npad=npad),
      mesh=mesh,
      out_type=jax.ShapeDtypeStruct((NC, npad), F32),
      scratch_types=[
          pltpu.VMEM((nchunks, K), jnp.int32),
          pltpu.VMEM((K,), F32),
          pltpu.VMEM((K,), F32),
          pltpu.VMEM_SHARED((npad,), F32),
      ],
  )
  return fn(dst3)


def _sc_agg(zs, src3, dst3, npad):
  nchunks = src3.shape[1]
  mesh = plsc.VectorSubcoreMesh(core_axis_name="c", subcore_axis_name="s")
  fn = pl.kernel(
      functools.partial(_agg_body, nchunks=nchunks, npad=npad),
      mesh=mesh,
      out_type=jax.ShapeDtypeStruct((NC, npad, 128), F32),
      scratch_types=[
          pltpu.VMEM((IB, K), jnp.int32),
          pltpu.VMEM((IB, K), jnp.int32),
          pltpu.VMEM((NBUF, K, 128), F32),
          pltpu.VMEM((16, 128), F32),
          pltpu.VMEM_SHARED((npad, 128), F32),
      ] + [pltpu.SemaphoreType.DMA] * NBUF,
  )
  return fn(zs, src3, dst3)


def _dinv_blk(degp_ref, i):
  d = degp_ref[:, pl.ds(i * BLK, BLK)]
  return lax.rsqrt(d[0] + d[1] + 1.0)


def _split(z):
  return z


def _mm_scale_body(x_ref, w_ref, degp_ref, o_ref):
  i = pl.program_id(0)
  dinv = _dinv_blk(degp_ref, i)
  z = jnp.dot(x_ref[...], w_ref[...], preferred_element_type=F32)
  o_ref[...] = z * dinv[:, None]


def _mid_body(agg_ref, zs_ref, degp_ref, b_ref, w_ref, o_ref):
  i = pl.program_id(0)
  dinv = _dinv_blk(degp_ref, i)[:, None]
  h = (agg_ref[0] + agg_ref[1] + zs_ref[...]) * dinv + b_ref[...]
  h = jnp.maximum(h, 0.0)
  o_ref[...] = jnp.dot(h, w_ref[...], preferred_element_type=F32) * dinv


def _pool_body(agg_ref, zs_ref, degp_ref, b_ref, batch_ref, wfc_ref, bfc_ref,
               o_ref, pacc, cacc, *, nblk, g):
  i = pl.program_id(0)
  dinv = _dinv_blk(degp_ref, i)[:, None]
  h = (agg_ref[0] + agg_ref[1] + zs_ref[...]) * dinv + b_ref[...]
  h = jnp.maximum(h, 0.0)
  bt = batch_ref[0, 0, :]
  mask = (bt[:, None] == lax.broadcasted_iota(jnp.int32, (BLK, g), 1))
  mask = mask.astype(F32)

  @pl.when(i == 0)
  def _():
    pacc[...] = jnp.zeros_like(pacc)
    cacc[...] = jnp.zeros_like(cacc)

  pacc[...] += lax.dot_general(mask, h, (((0,), (0,)), ((), ())),
                               preferred_element_type=F32)
  cacc[...] += jnp.sum(mask, axis=0)[None, :]

  @pl.when(i == nblk - 1)
  def _():
    cnt = jnp.maximum(cacc[...], 1.0).reshape(g, 1)
    pooled = pacc[...] / cnt
    o_ref[...] = (jnp.dot(pooled, wfc_ref[...], preferred_element_type=F32)
                  + bfc_ref[...])


def _tc_mm_scale(x_pad, w, degp, npad):
  nblk = npad // BLK
  return pl.pallas_call(
      _mm_scale_body,
      grid=(nblk,),
      in_specs=[
          pl.BlockSpec((BLK, 128), lambda i: (i, 0)),
          pl.BlockSpec((128, 128), lambda i: (0, 0)),
          pl.BlockSpec((NC, npad), lambda i: (0, 0)),
      ],
      out_specs=pl.BlockSpec((BLK, 128), lambda i: (i, 0)),
      out_shape=jax.ShapeDtypeStruct((npad, 128), F32),
  )(x_pad, w, degp)


def _tc_mid(agg, zs, degp, b, w, npad):
  nblk = npad // BLK
  return pl.pallas_call(
      _mid_body,
      grid=(nblk,),
      in_specs=[
          pl.BlockSpec((NC, BLK, 128), lambda i: (0, i, 0)),
          pl.BlockSpec((BLK, 128), lambda i: (i, 0)),
          pl.BlockSpec((NC, npad), lambda i: (0, 0)),
          pl.BlockSpec((1, 128), lambda i: (0, 0)),
          pl.BlockSpec((128, 128), lambda i: (0, 0)),
      ],
      out_specs=pl.BlockSpec((BLK, 128), lambda i: (i, 0)),
      out_shape=jax.ShapeDtypeStruct((npad, 128), F32),
  )(agg, zs, degp, b.reshape(1, 128), w)


def _tc_pool(agg, zs, degp, b, batch3, wfc, bfc, npad, g, c):
  nblk = npad // BLK
  return pl.pallas_call(
      functools.partial(_pool_body, nblk=nblk, g=g),
      grid=(nblk,),
      in_specs=[
          pl.BlockSpec((NC, BLK, 128), lambda i: (0, i, 0)),
          pl.BlockSpec((BLK, 128), lambda i: (i, 0)),
          pl.BlockSpec((NC, npad), lambda i: (0, 0)),
          pl.BlockSpec((1, 128), lambda i: (0, 0)),
          pl.BlockSpec((1, 1, BLK), lambda i: (i, 0, 0)),
          pl.BlockSpec((128, c), lambda i: (0, 0)),
          pl.BlockSpec((1, c), lambda i: (0, 0)),
      ],
      out_specs=pl.BlockSpec((g, c), lambda i: (0, 0)),
      out_shape=jax.ShapeDtypeStruct((g, c), F32),
      scratch_shapes=[
          pltpu.VMEM((g, 128), F32),
          pltpu.VMEM((1, g), F32),
      ],
  )(agg, zs, degp, b.reshape(1, 128), batch3, wfc, bfc.reshape(1, c))


def kernel(x, edge_index, batch, W1, b1, W2, b2, Wfc, bfc):
  n, d = x.shape
  e = edge_index.shape[1]
  g = 64
  c = Wfc.shape[1]

  npad = ((n + BLK) // BLK) * BLK
  ekc = NW * K * IB
  epad = -(-e // ekc) * ekc
  nch_agg = epad // (NW * K)
  nch_deg = epad // (NW * K)

  x_pad = jnp.pad(x, ((0, npad - n), (0, 0)))
  src_pad = jnp.concatenate(
      [edge_index[0], jnp.zeros((epad - e,), jnp.int32)])
  dst_pad = jnp.concatenate(
      [edge_index[1], jnp.full((epad - e,), n, jnp.int32)])
  src3a = src_pad.reshape(NW, nch_agg, K)
  dst3a = dst_pad.reshape(NW, nch_agg, K)
  dst3d = dst_pad.reshape(NW, nch_deg, K)
  batch3 = jnp.concatenate(
      [batch, jnp.full((npad - n,), -1, jnp.int32)]).reshape(-1, 1, BLK)

  degp = _sc_deg(dst3d, npad)
  zs1 = _tc_mm_scale(x_pad, W1, degp, npad)
  agg1 = _sc_agg(zs1, src3a, dst3a, npad)
  zs2 = _tc_mid(agg1, zs1, degp, b1, W2, npad)
  agg2 = _sc_agg(zs2, src3a, dst3a, npad)
  return _tc_pool(agg2, zs2, degp, b2, batch3, Wfc, bfc, npad, g, c)


# speedup vs baseline: 2.6922x; 2.6922x over previous
import functools

import jax
import jax.numpy as jnp
from jax import lax
from jax.experimental import pallas as pl
from jax.experimental.pallas import tpu as pltpu
from jax.experimental.pallas import tpu_sc as plsc

NC = 2
NS = 16
NW = NC * NS
K = 128
HH = 64
BLK = 256
NBUF = 2
IB = 40

F32 = jnp.float32


def _deg_body(dst_hbm, out_hbm, didx_all, ones_v, zb, acc, *, nchunks, npad):
  cid = lax.axis_index("c")
  sid = lax.axis_index("s")
  wid = cid * NS + sid
  rpt = npad // NS
  for c in range(8):
    zb[pl.ds(c * 16, 16)] = jnp.zeros((16,), F32)
    ones_v[pl.ds(c * 16, 16)] = jnp.full((16,), 1.0, F32)
  r0 = sid * rpt
  for t in range(rpt // K):
    pltpu.sync_copy(zb, acc.at[pl.ds(r0 + t * K, K)])
  pltpu.sync_copy(dst_hbm.at[wid], didx_all)
  plsc.subcore_barrier()

  def body(j, carry):
    pltpu.sync_copy(ones_v, acc.at[didx_all.at[j]], add=True)
    return carry

  lax.fori_loop(0, nchunks, body, 0)
  plsc.subcore_barrier()
  for t in range(rpt // K):
    pltpu.sync_copy(acc.at[pl.ds(r0 + t * K, K)],
                    out_hbm.at[cid, pl.ds(r0 + t * K, K)])


def _agg_body(zs_hbm, src_hbm, dst_hbm, out_hbm, sidx_blk, didx_blk,
              rows, zb, acc, *sems, nchunks, npad):
  cid = lax.axis_index("c")
  sid = lax.axis_index("s")
  rpt = npad // NS
  for i in range(16):
    for c in range(8):
      zb[i, pl.ds(c * 16, 16)] = jnp.zeros((16,), F32)
  r0 = sid * rpt
  for t in range(rpt // 16):
    pltpu.sync_copy(zb, acc.at[pl.ds(r0 + t * 16, 16)])
  wid = cid * NS + sid
  plsc.subcore_barrier()

  def outer(ib, carry):
    pltpu.sync_copy(src_hbm.at[wid, pl.ds(ib * IB, IB)], sidx_blk)
    pltpu.sync_copy(dst_hbm.at[wid, pl.ds(ib * IB, IB)], didx_blk)

    def body(i, carry2):
      handles = []
      for b in range(NBUF):
        c = i * NBUF + b
        handles.append(
            pltpu.async_copy(zs_hbm.at[sidx_blk.at[c]], rows.at[b], sems[b]))
      for b in range(NBUF):
        c = i * NBUF + b
        handles[b].wait()
        pltpu.sync_copy(rows.at[b], acc.at[didx_blk.at[c]], add=True)
      return carry2

    lax.fori_loop(0, IB // NBUF, body, 0)
    return carry

  lax.fori_loop(0, nchunks // IB, outer, 0)
  plsc.subcore_barrier()
  for t in range(rpt // K):
    pltpu.sync_copy(acc.at[pl.ds(r0 + t * K, K)],
                    out_hbm.at[cid, pl.ds(r0 + t * K, K)])


def _sc_deg(dst3, npad):
  nchunks = dst3.shape[1]
  mesh = plsc.VectorSubcoreMesh(core_axis_name="c", subcore_axis_name="s")
  fn = pl.kernel(
      functools.partial(_deg_body, nchunks=nchunks, npad=npad),
      mesh=mesh,
      out_type=jax.ShapeDtypeStruct((NC, npad), F32),
      scratch_types=[
          pltpu.VMEM((nchunks, K), jnp.int32),
          pltpu.VMEM((K,), F32),
          pltpu.VMEM((K,), F32),
          pltpu.VMEM_SHARED((npad,), F32),
      ],
  )
  return fn(dst3)


def _sc_agg(zs, src3, dst3, npad):
  nchunks = src3.shape[1]
  mesh = plsc.VectorSubcoreMesh(core_axis_name="c", subcore_axis_name="s")
  fn = pl.kernel(
      functools.partial(_agg_body, nchunks=nchunks, npad=npad),
      mesh=mesh,
      out_type=jax.ShapeDtypeStruct((NC, npad, 128), F32),
      scratch_types=[
          pltpu.VMEM((IB, K), jnp.int32),
          pltpu.VMEM((IB, K), jnp.int32),
          pltpu.VMEM((NBUF, K, 128), F32),
          pltpu.VMEM((16, 128), F32),
          pltpu.VMEM_SHARED((npad, 128), F32),
      ] + [pltpu.SemaphoreType.DMA] * NBUF,
  )
  return fn(zs, src3, dst3)


def _dinv_blk(degp_ref, i):
  d = degp_ref[:, pl.ds(i * BLK, BLK)]
  return lax.rsqrt(d[0] + d[1] + 1.0)


def _split(z):
  return z


def _mm_scale_body(x_ref, w_ref, degp_ref, o_ref):
  i = pl.program_id(0)
  dinv = _dinv_blk(degp_ref, i)
  z = jnp.dot(x_ref[...], w_ref[...], preferred_element_type=F32)
  o_ref[...] = z * dinv[:, None]


def _mid_body(agg_ref, zs_ref, degp_ref, b_ref, w_ref, o_ref):
  i = pl.program_id(0)
  dinv = _dinv_blk(degp_ref, i)[:, None]
  h = (agg_ref[0] + agg_ref[1] + zs_ref[...]) * dinv + b_ref[...]
  h = jnp.maximum(h, 0.0)
  o_ref[...] = jnp.dot(h, w_ref[...], preferred_element_type=F32) * dinv


def _pool_body(agg_ref, zs_ref, degp_ref, b_ref, batch_ref, wfc_ref, bfc_ref,
               o_ref, pacc, cacc, *, nblk, g):
  i = pl.program_id(0)
  dinv = _dinv_blk(degp_ref, i)[:, None]
  h = (agg_ref[0] + agg_ref[1] + zs_ref[...]) * dinv + b_ref[...]
  h = jnp.maximum(h, 0.0)
  bt = batch_ref[0, 0, :]
  mask = (bt[:, None] == lax.broadcasted_iota(jnp.int32, (BLK, g), 1))
  mask = mask.astype(F32)

  @pl.when(i == 0)
  def _():
    pacc[...] = jnp.zeros_like(pacc)
    cacc[...] = jnp.zeros_like(cacc)

  pacc[...] += lax.dot_general(mask, h, (((0,), (0,)), ((), ())),
                               preferred_element_type=F32)
  cacc[...] += jnp.sum(mask, axis=0)[None, :]

  @pl.when(i == nblk - 1)
  def _():
    cnt = jnp.maximum(cacc[...], 1.0).reshape(g, 1)
    pooled = pacc[...] / cnt
    o_ref[...] = (jnp.dot(pooled, wfc_ref[...], preferred_element_type=F32)
                  + bfc_ref[...])


def _tc_mm_scale(x_pad, w, degp, npad):
  nblk = npad // BLK
  return pl.pallas_call(
      _mm_scale_body,
      grid=(nblk,),
      in_specs=[
          pl.BlockSpec((BLK, 128), lambda i: (i, 0)),
          pl.BlockSpec((128, 128), lambda i: (0, 0)),
          pl.BlockSpec((NC, npad), lambda i: (0, 0)),
      ],
      out_specs=pl.BlockSpec((BLK, 128), lambda i: (i, 0)),
      out_shape=jax.ShapeDtypeStruct((npad, 128), F32),
  )(x_pad, w, degp)


def _tc_mid(agg, zs, degp, b, w, npad):
  nblk = npad // BLK
  return pl.pallas_call(
      _mid_body,
      grid=(nblk,),
      in_specs=[
          pl.BlockSpec((NC, BLK, 128), lambda i: (0, i, 0)),
          pl.BlockSpec((BLK, 128), lambda i: (i, 0)),
          pl.BlockSpec((NC, npad), lambda i: (0, 0)),
          pl.BlockSpec((1, 128), lambda i: (0, 0)),
          pl.BlockSpec((128, 128), lambda i: (0, 0)),
      ],
      out_specs=pl.BlockSpec((BLK, 128), lambda i: (i, 0)),
      out_shape=jax.ShapeDtypeStruct((npad, 128), F32),
  )(agg, zs, degp, b.reshape(1, 128), w)


def _tc_pool(agg, zs, degp, b, batch3, wfc, bfc, npad, g, c):
  nblk = npad // BLK
  return pl.pallas_call(
      functools.partial(_pool_body, nblk=nblk, g=g),
      grid=(nblk,),
      in_specs=[
          pl.BlockSpec((NC, BLK, 128), lambda i: (0, i, 0)),
          pl.BlockSpec((BLK, 128), lambda i: (i, 0)),
          pl.BlockSpec((NC, npad), lambda i: (0, 0)),
          pl.BlockSpec((1, 128), lambda i: (0, 0)),
          pl.BlockSpec((1, 1, BLK), lambda i: (i, 0, 0)),
          pl.BlockSpec((128, c), lambda i: (0, 0)),
          pl.BlockSpec((1, c), lambda i: (0, 0)),
      ],
      out_specs=pl.BlockSpec((g, c), lambda i: (0, 0)),
      out_shape=jax.ShapeDtypeStruct((g, c), F32),
      scratch_shapes=[
          pltpu.VMEM((g, 128), F32),
          pltpu.VMEM((1, g), F32),
      ],
  )(agg, zs, degp, b.reshape(1, 128), batch3, wfc, bfc.reshape(1, c))


def kernel(x, edge_index, batch, W1, b1, W2, b2, Wfc, bfc):
  n, d = x.shape
  e = edge_index.shape[1]
  g = 64
  c = Wfc.shape[1]

  npad = ((n + BLK) // BLK) * BLK
  ekc = NW * K * IB
  epad = -(-e // ekc) * ekc
  nch_agg = epad // (NW * K)
  nch_deg = epad // (NW * K)

  x_pad = jnp.pad(x, ((0, npad - n), (0, 0)))
  spread = n + jnp.arange(epad - e, dtype=jnp.int32) % (npad - n)
  src_pad = jnp.concatenate([edge_index[0], spread])
  dst_pad = jnp.concatenate([edge_index[1], spread])
  src3a = src_pad.reshape(NW, nch_agg, K)
  dst3a = dst_pad.reshape(NW, nch_agg, K)
  dst3d = dst_pad.reshape(NW, nch_deg, K)
  batch3 = jnp.concatenate(
      [batch, jnp.full((npad - n,), -1, jnp.int32)]).reshape(-1, 1, BLK)

  degp = _sc_deg(dst3d, npad)
  zs1 = _tc_mm_scale(x_pad, W1, degp, npad)
  agg1 = _sc_agg(zs1, src3a, dst3a, npad)
  zs2 = _tc_mid(agg1, zs1, degp, b1, W2, npad)
  agg2 = _sc_agg(zs2, src3a, dst3a, npad)
  return _tc_pool(agg2, zs2, degp, b2, batch3, Wfc, bfc, npad, g, c)


# speedup vs baseline: 2.9731x; 1.1043x over previous
import functools

import jax
import jax.numpy as jnp
from jax import lax
from jax.experimental import pallas as pl
from jax.experimental.pallas import tpu as pltpu
from jax.experimental.pallas import tpu_sc as plsc

NC = 2
NS = 16
NW = NC * NS
K = 128
HH = 64
BLK = 256
NBUF = 2
IB = 40

F32 = jnp.float32


def _deg_body(dst_hbm, out_hbm, didx_all, ones_v, zb, acc, *, nchunks, npad):
  cid = lax.axis_index("c")
  sid = lax.axis_index("s")
  wid = cid * NS + sid
  rpt = npad // NS
  for c in range(8):
    zb[pl.ds(c * 16, 16)] = jnp.zeros((16,), F32)
    ones_v[pl.ds(c * 16, 16)] = jnp.full((16,), 1.0, F32)
  r0 = sid * rpt
  for t in range(rpt // K):
    pltpu.sync_copy(zb, acc.at[pl.ds(r0 + t * K, K)])
  pltpu.sync_copy(dst_hbm.at[wid], didx_all)
  plsc.subcore_barrier()

  def body(j, carry):
    pltpu.sync_copy(ones_v, acc.at[didx_all.at[j]], add=True)
    return carry

  lax.fori_loop(0, nchunks, body, 0)
  plsc.subcore_barrier()
  for t in range(rpt // K):
    pltpu.sync_copy(acc.at[pl.ds(r0 + t * K, K)],
                    out_hbm.at[cid, pl.ds(r0 + t * K, K)])


def _agg_body(zs_hbm, src_hbm, dst_hbm, out_hbm, sidx_blk, didx_blk,
              rows, zb, acc, *sems, nchunks, npad):
  cid = lax.axis_index("c")
  sid = lax.axis_index("s")
  rpt = npad // NS
  for i in range(16):
    for c in range(8):
      zb[i, pl.ds(c * 16, 16)] = jnp.zeros((16,), F32)
  r0 = sid * rpt
  for t in range(rpt // 16):
    pltpu.sync_copy(zb, acc.at[pl.ds(r0 + t * 16, 16)])
  wid = cid * NS + sid
  plsc.subcore_barrier()
  gsem = sems[:NBUF]
  ssem = sems[NBUF:]

  def gissue(c, b):
    pltpu.async_copy(zs_hbm.at[sidx_blk.at[c]], rows.at[b], gsem[b])

  def gwait(c, b):
    pltpu.make_async_copy(zs_hbm.at[sidx_blk.at[c]], rows.at[b],
                          gsem[b]).wait()

  def sissue(c, b):
    pltpu.async_copy(rows.at[b], acc.at[didx_blk.at[c]], ssem[b], add=True)

  def swait(c, b):
    pltpu.make_async_copy(rows.at[b], acc.at[didx_blk.at[c]],
                          ssem[b]).wait()

  def outer(ib, carry):
    pltpu.sync_copy(src_hbm.at[wid, pl.ds(ib * IB, IB)], sidx_blk)
    pltpu.sync_copy(dst_hbm.at[wid, pl.ds(ib * IB, IB)], didx_blk)
    gissue(0, 0)

    def pair(i, carry2):
      c0 = 2 * i
      gwait(c0, 0)
      sissue(c0, 0)

      @pl.when(i > 0)
      def _():
        swait(c0 - 1, 1)

      gissue(c0 + 1, 1)
      gwait(c0 + 1, 1)
      sissue(c0 + 1, 1)
      swait(c0, 0)

      @pl.when(c0 + 2 < IB)
      def _():
        gissue(c0 + 2, 0)

      return carry2

    lax.fori_loop(0, IB // 2, pair, 0)
    swait(IB - 1, 1)
    return carry

  lax.fori_loop(0, nchunks // IB, outer, 0)
  plsc.subcore_barrier()
  for t in range(rpt // K):
    pltpu.sync_copy(acc.at[pl.ds(r0 + t * K, K)],
                    out_hbm.at[cid, pl.ds(r0 + t * K, K)])


def _sc_deg(dst3, npad):
  nchunks = dst3.shape[1]
  mesh = plsc.VectorSubcoreMesh(core_axis_name="c", subcore_axis_name="s")
  fn = pl.kernel(
      functools.partial(_deg_body, nchunks=nchunks, npad=npad),
      mesh=mesh,
      out_type=jax.ShapeDtypeStruct((NC, npad), F32),
      scratch_types=[
          pltpu.VMEM((nchunks, K), jnp.int32),
          pltpu.VMEM((K,), F32),
          pltpu.VMEM((K,), F32),
          pltpu.VMEM_SHARED((npad,), F32),
      ],
  )
  return fn(dst3)


def _sc_agg(zs, src3, dst3, npad):
  nchunks = src3.shape[1]
  mesh = plsc.VectorSubcoreMesh(core_axis_name="c", subcore_axis_name="s")
  fn = pl.kernel(
      functools.partial(_agg_body, nchunks=nchunks, npad=npad),
      mesh=mesh,
      out_type=jax.ShapeDtypeStruct((NC, npad, 128), F32),
      scratch_types=[
          pltpu.VMEM((IB, K), jnp.int32),
          pltpu.VMEM((IB, K), jnp.int32),
          pltpu.VMEM((NBUF, K, 128), F32),
          pltpu.VMEM((16, 128), F32),
          pltpu.VMEM_SHARED((npad, 128), F32),
      ] + [pltpu.SemaphoreType.DMA] * (2 * NBUF),
  )
  return fn(zs, src3, dst3)


def _dinv_blk(degp_ref, i):
  d = degp_ref[:, pl.ds(i * BLK, BLK)]
  return lax.rsqrt(d[0] + d[1] + 1.0)


def _split(z):
  return z


def _mm_scale_body(x_ref, w_ref, degp_ref, o_ref):
  i = pl.program_id(0)
  dinv = _dinv_blk(degp_ref, i)
  z = jnp.dot(x_ref[...], w_ref[...], preferred_element_type=F32)
  o_ref[...] = z * dinv[:, None]


def _mid_body(agg_ref, zs_ref, degp_ref, b_ref, w_ref, o_ref):
  i = pl.program_id(0)
  dinv = _dinv_blk(degp_ref, i)[:, None]
  h = (agg_ref[0] + agg_ref[1] + zs_ref[...]) * dinv + b_ref[...]
  h = jnp.maximum(h, 0.0)
  o_ref[...] = jnp.dot(h, w_ref[...], preferred_element_type=F32) * dinv


def _pool_body(agg_ref, zs_ref, degp_ref, b_ref, batch_ref, wfc_ref, bfc_ref,
               o_ref, pacc, cacc, *, nblk, g):
  i = pl.program_id(0)
  dinv = _dinv_blk(degp_ref, i)[:, None]
  h = (agg_ref[0] + agg_ref[1] + zs_ref[...]) * dinv + b_ref[...]
  h = jnp.maximum(h, 0.0)
  bt = batch_ref[0, 0, :]
  mask = (bt[:, None] == lax.broadcasted_iota(jnp.int32, (BLK, g), 1))
  mask = mask.astype(F32)

  @pl.when(i == 0)
  def _():
    pacc[...] = jnp.zeros_like(pacc)
    cacc[...] = jnp.zeros_like(cacc)

  pacc[...] += lax.dot_general(mask, h, (((0,), (0,)), ((), ())),
                               preferred_element_type=F32)
  cacc[...] += jnp.sum(mask, axis=0)[None, :]

  @pl.when(i == nblk - 1)
  def _():
    cnt = jnp.maximum(cacc[...], 1.0).reshape(g, 1)
    pooled = pacc[...] / cnt
    o_ref[...] = (jnp.dot(pooled, wfc_ref[...], preferred_element_type=F32)
                  + bfc_ref[...])


def _tc_mm_scale(x_pad, w, degp, npad):
  nblk = npad // BLK
  return pl.pallas_call(
      _mm_scale_body,
      grid=(nblk,),
      in_specs=[
          pl.BlockSpec((BLK, 128), lambda i: (i, 0)),
          pl.BlockSpec((128, 128), lambda i: (0, 0)),
          pl.BlockSpec((NC, npad), lambda i: (0, 0)),
      ],
      out_specs=pl.BlockSpec((BLK, 128), lambda i: (i, 0)),
      out_shape=jax.ShapeDtypeStruct((npad, 128), F32),
  )(x_pad, w, degp)


def _tc_mid(agg, zs, degp, b, w, npad):
  nblk = npad // BLK
  return pl.pallas_call(
      _mid_body,
      grid=(nblk,),
      in_specs=[
          pl.BlockSpec((NC, BLK, 128), lambda i: (0, i, 0)),
          pl.BlockSpec((BLK, 128), lambda i: (i, 0)),
          pl.BlockSpec((NC, npad), lambda i: (0, 0)),
          pl.BlockSpec((1, 128), lambda i: (0, 0)),
          pl.BlockSpec((128, 128), lambda i: (0, 0)),
      ],
      out_specs=pl.BlockSpec((BLK, 128), lambda i: (i, 0)),
      out_shape=jax.ShapeDtypeStruct((npad, 128), F32),
  )(agg, zs, degp, b.reshape(1, 128), w)


def _tc_pool(agg, zs, degp, b, batch3, wfc, bfc, npad, g, c):
  nblk = npad // BLK
  return pl.pallas_call(
      functools.partial(_pool_body, nblk=nblk, g=g),
      grid=(nblk,),
      in_specs=[
          pl.BlockSpec((NC, BLK, 128), lambda i: (0, i, 0)),
          pl.BlockSpec((BLK, 128), lambda i: (i, 0)),
          pl.BlockSpec((NC, npad), lambda i: (0, 0)),
          pl.BlockSpec((1, 128), lambda i: (0, 0)),
          pl.BlockSpec((1, 1, BLK), lambda i: (i, 0, 0)),
          pl.BlockSpec((128, c), lambda i: (0, 0)),
          pl.BlockSpec((1, c), lambda i: (0, 0)),
      ],
      out_specs=pl.BlockSpec((g, c), lambda i: (0, 0)),
      out_shape=jax.ShapeDtypeStruct((g, c), F32),
      scratch_shapes=[
          pltpu.VMEM((g, 128), F32),
          pltpu.VMEM((1, g), F32),
      ],
  )(agg, zs, degp, b.reshape(1, 128), batch3, wfc, bfc.reshape(1, c))


def kernel(x, edge_index, batch, W1, b1, W2, b2, Wfc, bfc):
  n, d = x.shape
  e = edge_index.shape[1]
  g = 64
  c = Wfc.shape[1]

  npad = ((n + BLK) // BLK) * BLK
  ekc = NW * K * IB
  epad = -(-e // ekc) * ekc
  nch_agg = epad // (NW * K)
  nch_deg = epad // (NW * K)

  x_pad = jnp.pad(x, ((0, npad - n), (0, 0)))
  spread = n + jnp.arange(epad - e, dtype=jnp.int32) % (npad - n)
  src_pad = jnp.concatenate([edge_index[0], spread])
  dst_pad = jnp.concatenate([edge_index[1], spread])
  src3a = src_pad.reshape(NW, nch_agg, K)
  dst3a = dst_pad.reshape(NW, nch_agg, K)
  dst3d = dst_pad.reshape(NW, nch_deg, K)
  batch3 = jnp.concatenate(
      [batch, jnp.full((npad - n,), -1, jnp.int32)]).reshape(-1, 1, BLK)

  degp = _sc_deg(dst3d, npad)
  zs1 = _tc_mm_scale(x_pad, W1, degp, npad)
  agg1 = _sc_agg(zs1, src3a, dst3a, npad)
  zs2 = _tc_mid(agg1, zs1, degp, b1, W2, npad)
  agg2 = _sc_agg(zs2, src3a, dst3a, npad)
  return _tc_pool(agg2, zs2, degp, b2, batch3, Wfc, bfc, npad, g, c)


# speedup vs baseline: 3.3412x; 1.1238x over previous
import functools

import jax
import jax.numpy as jnp
from jax import lax
from jax.experimental import pallas as pl
from jax.experimental.pallas import tpu as pltpu
from jax.experimental.pallas import tpu_sc as plsc

NC = 2
NS = 16
NW = NC * NS
K = 128
HH = 64
BLK = 256
NBUF = 2
IB = 40

F32 = jnp.float32


def _deg_body(dst_hbm, out_hbm, didx_all, ones_v, zb, acc, *, nchunks, npad):
  cid = lax.axis_index("c")
  sid = lax.axis_index("s")
  wid = cid * NS + sid
  rpt = npad // NS
  for c in range(8):
    zb[pl.ds(c * 16, 16)] = jnp.zeros((16,), F32)
    ones_v[pl.ds(c * 16, 16)] = jnp.full((16,), 1.0, F32)
  r0 = sid * rpt
  for t in range(rpt // K):
    pltpu.sync_copy(zb, acc.at[pl.ds(r0 + t * K, K)])
  pltpu.sync_copy(dst_hbm.at[wid], didx_all)
  plsc.subcore_barrier()

  def body(j, carry):
    pltpu.sync_copy(ones_v, acc.at[didx_all.at[j]], add=True)
    return carry

  lax.fori_loop(0, nchunks, body, 0)
  plsc.subcore_barrier()
  for t in range(rpt // K):
    pltpu.sync_copy(acc.at[pl.ds(r0 + t * K, K)],
                    out_hbm.at[cid, pl.ds(r0 + t * K, K)])


def _agg_body(zs_hbm, src_hbm, dst_hbm, out_hbm, sidx_blk, didx_blk,
              rows, zb, acc, *sems, nchunks, npad):
  cid = lax.axis_index("c")
  sid = lax.axis_index("s")
  rpt = npad // NS
  for i in range(16):
    for c in range(8):
      zb[i, pl.ds(c * 16, 16)] = jnp.zeros((16,), F32)
  r0 = sid * rpt
  for t in range(rpt // 16):
    pltpu.sync_copy(zb, acc.at[pl.ds(r0 + t * 16, 16)])
  wid = cid * NS + sid
  plsc.subcore_barrier()
  gsem = sems[:NBUF]
  ssem = sems[NBUF:]

  def gissue(c, b):
    pltpu.async_copy(zs_hbm.at[sidx_blk.at[c]], rows.at[b], gsem[b])

  def gwait(c, b):
    pltpu.make_async_copy(zs_hbm.at[sidx_blk.at[c]], rows.at[b],
                          gsem[b]).wait()

  def sissue(c, b):
    pltpu.async_copy(rows.at[b], acc.at[didx_blk.at[c]], ssem[b], add=True)

  def swait(c, b):
    pltpu.make_async_copy(rows.at[b], acc.at[didx_blk.at[c]],
                          ssem[b]).wait()

  def outer(ib, carry):
    pltpu.sync_copy(src_hbm.at[wid, pl.ds(ib * IB, IB)], sidx_blk)
    pltpu.sync_copy(dst_hbm.at[wid, pl.ds(ib * IB, IB)], didx_blk)
    gissue(0, 0)
    gissue(1, 1)

    def pair(i, carry2):
      c0 = 2 * i
      for b in range(2):
        c = c0 + b
        gwait(c, b)
        sissue(c, b)
        swait(c, b)

        @pl.when(c + 2 < IB)
        def _():
          gissue(c + 2, b)

      return carry2

    lax.fori_loop(0, IB // 2, pair, 0)
    return carry

  lax.fori_loop(0, nchunks // IB, outer, 0)
  plsc.subcore_barrier()
  for t in range(rpt // K):
    pltpu.sync_copy(acc.at[pl.ds(r0 + t * K, K)],
                    out_hbm.at[cid, pl.ds(r0 + t * K, K)])


def _sc_deg(dst3, npad):
  nchunks = dst3.shape[1]
  mesh = plsc.VectorSubcoreMesh(core_axis_name="c", subcore_axis_name="s")
  fn = pl.kernel(
      functools.partial(_deg_body, nchunks=nchunks, npad=npad),
      mesh=mesh,
      out_type=jax.ShapeDtypeStruct((NC, npad), F32),
      scratch_types=[
          pltpu.VMEM((nchunks, K), jnp.int32),
          pltpu.VMEM((K,), F32),
          pltpu.VMEM((K,), F32),
          pltpu.VMEM_SHARED((npad,), F32),
      ],
  )
  return fn(dst3)


def _sc_agg(zs, src3, dst3, npad):
  nchunks = src3.shape[1]
  mesh = plsc.VectorSubcoreMesh(core_axis_name="c", subcore_axis_name="s")
  fn = pl.kernel(
      functools.partial(_agg_body, nchunks=nchunks, npad=npad),
      mesh=mesh,
      out_type=jax.ShapeDtypeStruct((NC, npad, 128), F32),
      scratch_types=[
          pltpu.VMEM((IB, K), jnp.int32),
          pltpu.VMEM((IB, K), jnp.int32),
          pltpu.VMEM((NBUF, K, 128), F32),
          pltpu.VMEM((16, 128), F32),
          pltpu.VMEM_SHARED((npad, 128), F32),
      ] + [pltpu.SemaphoreType.DMA] * (2 * NBUF),
  )
  return fn(zs, src3, dst3)


def _dinv_blk(degp_ref, i):
  d = degp_ref[:, pl.ds(i * BLK, BLK)]
  return lax.rsqrt(d[0] + d[1] + 1.0)


def _split(z):
  return z


def _mm_scale_body(x_ref, w_ref, degp_ref, o_ref):
  i = pl.program_id(0)
  dinv = _dinv_blk(degp_ref, i)
  z = jnp.dot(x_ref[...], w_ref[...], preferred_element_type=F32)
  o_ref[...] = z * dinv[:, None]


def _mid_body(agg_ref, zs_ref, degp_ref, b_ref, w_ref, o_ref):
  i = pl.program_id(0)
  dinv = _dinv_blk(degp_ref, i)[:, None]
  h = (agg_ref[0] + agg_ref[1] + zs_ref[...]) * dinv + b_ref[...]
  h = jnp.maximum(h, 0.0)
  o_ref[...] = jnp.dot(h, w_ref[...], preferred_element_type=F32) * dinv


def _pool_body(agg_ref, zs_ref, degp_ref, b_ref, batch_ref, wfc_ref, bfc_ref,
               o_ref, pacc, cacc, *, nblk, g):
  i = pl.program_id(0)
  dinv = _dinv_blk(degp_ref, i)[:, None]
  h = (agg_ref[0] + agg_ref[1] + zs_ref[...]) * dinv + b_ref[...]
  h = jnp.maximum(h, 0.0)
  bt = batch_ref[0, 0, :]
  mask = (bt[:, None] == lax.broadcasted_iota(jnp.int32, (BLK, g), 1))
  mask = mask.astype(F32)

  @pl.when(i == 0)
  def _():
    pacc[...] = jnp.zeros_like(pacc)
    cacc[...] = jnp.zeros_like(cacc)

  pacc[...] += lax.dot_general(mask, h, (((0,), (0,)), ((), ())),
                               preferred_element_type=F32)
  cacc[...] += jnp.sum(mask, axis=0)[None, :]

  @pl.when(i == nblk - 1)
  def _():
    cnt = jnp.maximum(cacc[...], 1.0).reshape(g, 1)
    pooled = pacc[...] / cnt
    o_ref[...] = (jnp.dot(pooled, wfc_ref[...], preferred_element_type=F32)
                  + bfc_ref[...])


def _tc_mm_scale(x_pad, w, degp, npad):
  nblk = npad // BLK
  return pl.pallas_call(
      _mm_scale_body,
      grid=(nblk,),
      in_specs=[
          pl.BlockSpec((BLK, 128), lambda i: (i, 0)),
          pl.BlockSpec((128, 128), lambda i: (0, 0)),
          pl.BlockSpec((NC, npad), lambda i: (0, 0)),
      ],
      out_specs=pl.BlockSpec((BLK, 128), lambda i: (i, 0)),
      out_shape=jax.ShapeDtypeStruct((npad, 128), F32),
  )(x_pad, w, degp)


def _tc_mid(agg, zs, degp, b, w, npad):
  nblk = npad // BLK
  return pl.pallas_call(
      _mid_body,
      grid=(nblk,),
      in_specs=[
          pl.BlockSpec((NC, BLK, 128), lambda i: (0, i, 0)),
          pl.BlockSpec((BLK, 128), lambda i: (i, 0)),
          pl.BlockSpec((NC, npad), lambda i: (0, 0)),
          pl.BlockSpec((1, 128), lambda i: (0, 0)),
          pl.BlockSpec((128, 128), lambda i: (0, 0)),
      ],
      out_specs=pl.BlockSpec((BLK, 128), lambda i: (i, 0)),
      out_shape=jax.ShapeDtypeStruct((npad, 128), F32),
  )(agg, zs, degp, b.reshape(1, 128), w)


def _tc_pool(agg, zs, degp, b, batch3, wfc, bfc, npad, g, c):
  nblk = npad // BLK
  return pl.pallas_call(
      functools.partial(_pool_body, nblk=nblk, g=g),
      grid=(nblk,),
      in_specs=[
          pl.BlockSpec((NC, BLK, 128), lambda i: (0, i, 0)),
          pl.BlockSpec((BLK, 128), lambda i: (i, 0)),
          pl.BlockSpec((NC, npad), lambda i: (0, 0)),
          pl.BlockSpec((1, 128), lambda i: (0, 0)),
          pl.BlockSpec((1, 1, BLK), lambda i: (i, 0, 0)),
          pl.BlockSpec((128, c), lambda i: (0, 0)),
          pl.BlockSpec((1, c), lambda i: (0, 0)),
      ],
      out_specs=pl.BlockSpec((g, c), lambda i: (0, 0)),
      out_shape=jax.ShapeDtypeStruct((g, c), F32),
      scratch_shapes=[
          pltpu.VMEM((g, 128), F32),
          pltpu.VMEM((1, g), F32),
      ],
  )(agg, zs, degp, b.reshape(1, 128), batch3, wfc, bfc.reshape(1, c))


def kernel(x, edge_index, batch, W1, b1, W2, b2, Wfc, bfc):
  n, d = x.shape
  e = edge_index.shape[1]
  g = 64
  c = Wfc.shape[1]

  npad = ((n + BLK) // BLK) * BLK
  ekc = NW * K * IB
  epad = -(-e // ekc) * ekc
  nch_agg = epad // (NW * K)
  nch_deg = epad // (NW * K)

  x_pad = jnp.pad(x, ((0, npad - n), (0, 0)))
  spread = n + jnp.arange(epad - e, dtype=jnp.int32) % (npad - n)
  src_pad = jnp.concatenate([edge_index[0], spread])
  dst_pad = jnp.concatenate([edge_index[1], spread])
  src3a = src_pad.reshape(NW, nch_agg, K)
  dst3a = dst_pad.reshape(NW, nch_agg, K)
  dst3d = dst_pad.reshape(NW, nch_deg, K)
  batch3 = jnp.concatenate(
      [batch, jnp.full((npad - n,), -1, jnp.int32)]).reshape(-1, 1, BLK)

  degp = _sc_deg(dst3d, npad)
  zs1 = _tc_mm_scale(x_pad, W1, degp, npad)
  agg1 = _sc_agg(zs1, src3a, dst3a, npad)
  zs2 = _tc_mid(agg1, zs1, degp, b1, W2, npad)
  agg2 = _sc_agg(zs2, src3a, dst3a, npad)
  return _tc_pool(agg2, zs2, degp, b2, batch3, Wfc, bfc, npad, g, c)


# speedup vs baseline: 3.3491x; 1.0024x over previous
import functools

import jax
import jax.numpy as jnp
from jax import lax
from jax.experimental import pallas as pl
from jax.experimental.pallas import tpu as pltpu
from jax.experimental.pallas import tpu_sc as plsc

NC = 2
NS = 16
NW = NC * NS
K = 64
HH = 64
BLK = 256
NBUF = 4
IB = 32

F32 = jnp.float32


def _deg_body(dst_hbm, out_hbm, didx_all, ones_v, zb, acc, *, nchunks, npad):
  cid = lax.axis_index("c")
  sid = lax.axis_index("s")
  wid = cid * NS + sid
  rpt = npad // NS
  for c in range(8):
    zb[pl.ds(c * 16, 16)] = jnp.zeros((16,), F32)
  for c in range(K // 16):
    ones_v[pl.ds(c * 16, 16)] = jnp.full((16,), 1.0, F32)
  r0 = sid * rpt
  for t in range(rpt // 128):
    pltpu.sync_copy(zb, acc.at[pl.ds(r0 + t * 128, 128)])
  pltpu.sync_copy(dst_hbm.at[wid], didx_all)
  plsc.subcore_barrier()

  def body(j, carry):
    pltpu.sync_copy(ones_v, acc.at[didx_all.at[j]], add=True)
    return carry

  lax.fori_loop(0, nchunks, body, 0)
  plsc.subcore_barrier()
  for t in range(rpt // 128):
    pltpu.sync_copy(acc.at[pl.ds(r0 + t * 128, 128)],
                    out_hbm.at[cid, pl.ds(r0 + t * 128, 128)])


def _agg_body(zs_hbm, src_hbm, dst_hbm, out_hbm, sidx_blk, didx_blk,
              rows, zb, acc, *sems, nchunks, npad):
  cid = lax.axis_index("c")
  sid = lax.axis_index("s")
  rpt = npad // NS
  for i in range(16):
    for c in range(8):
      zb[i, pl.ds(c * 16, 16)] = jnp.zeros((16,), F32)
  r0 = sid * rpt
  for t in range(rpt // 16):
    pltpu.sync_copy(zb, acc.at[pl.ds(r0 + t * 16, 16)])
  wid = cid * NS + sid
  plsc.subcore_barrier()
  gsem = sems[:NBUF]
  ssem = sems[NBUF:]

  def gissue(c, b):
    pltpu.async_copy(zs_hbm.at[sidx_blk.at[c]], rows.at[b], gsem[b])

  def gwait(c, b):
    pltpu.make_async_copy(zs_hbm.at[sidx_blk.at[c]], rows.at[b],
                          gsem[b]).wait()

  def sissue(c, b):
    pltpu.async_copy(rows.at[b], acc.at[didx_blk.at[c]], ssem[b], add=True)

  def swait(c, b):
    pltpu.make_async_copy(rows.at[b], acc.at[didx_blk.at[c]],
                          ssem[b]).wait()

  def outer(ib, carry):
    pltpu.sync_copy(src_hbm.at[wid, pl.ds(ib * IB, IB)], sidx_blk)
    pltpu.sync_copy(dst_hbm.at[wid, pl.ds(ib * IB, IB)], didx_blk)
    for b in range(NBUF):
      gissue(b, b)

    def group(i, carry2):
      c0 = NBUF * i
      for b in range(NBUF):
        c = c0 + b
        gwait(c, b)
        sissue(c, b)
        swait(c, b)

        @pl.when(c + NBUF < IB)
        def _():
          gissue(c + NBUF, b)

      return carry2

    lax.fori_loop(0, IB // NBUF, group, 0)
    return carry

  lax.fori_loop(0, nchunks // IB, outer, 0)
  plsc.subcore_barrier()
  for t in range(rpt // 128):
    pltpu.sync_copy(acc.at[pl.ds(r0 + t * 128, 128)],
                    out_hbm.at[cid, pl.ds(r0 + t * 128, 128)])


def _sc_deg(dst3, npad):
  nchunks = dst3.shape[1]
  mesh = plsc.VectorSubcoreMesh(core_axis_name="c", subcore_axis_name="s")
  fn = pl.kernel(
      functools.partial(_deg_body, nchunks=nchunks, npad=npad),
      mesh=mesh,
      out_type=jax.ShapeDtypeStruct((NC, npad), F32),
      scratch_types=[
          pltpu.VMEM((nchunks, K), jnp.int32),
          pltpu.VMEM((K,), F32),
          pltpu.VMEM((128,), F32),
          pltpu.VMEM_SHARED((npad,), F32),
      ],
  )
  return fn(dst3)


def _sc_agg(zs, src3, dst3, npad):
  nchunks = src3.shape[1]
  mesh = plsc.VectorSubcoreMesh(core_axis_name="c", subcore_axis_name="s")
  fn = pl.kernel(
      functools.partial(_agg_body, nchunks=nchunks, npad=npad),
      mesh=mesh,
      out_type=jax.ShapeDtypeStruct((NC, npad, 128), F32),
      scratch_types=[
          pltpu.VMEM((IB, K), jnp.int32),
          pltpu.VMEM((IB, K), jnp.int32),
          pltpu.VMEM((NBUF, K, 128), F32),
          pltpu.VMEM((16, 128), F32),
          pltpu.VMEM_SHARED((npad, 128), F32),
      ] + [pltpu.SemaphoreType.DMA] * (2 * NBUF),
  )
  return fn(zs, src3, dst3)


def _dinv_blk(degp_ref, i):
  d = degp_ref[:, pl.ds(i * BLK, BLK)]
  return lax.rsqrt(d[0] + d[1] + 1.0)


def _split(z):
  return z


def _mm_scale_body(x_ref, w_ref, degp_ref, o_ref):
  i = pl.program_id(0)
  dinv = _dinv_blk(degp_ref, i)
  z = jnp.dot(x_ref[...], w_ref[...], preferred_element_type=F32)
  o_ref[...] = z * dinv[:, None]


def _mid_body(agg_ref, zs_ref, degp_ref, b_ref, w_ref, o_ref):
  i = pl.program_id(0)
  dinv = _dinv_blk(degp_ref, i)[:, None]
  h = (agg_ref[0] + agg_ref[1] + zs_ref[...]) * dinv + b_ref[...]
  h = jnp.maximum(h, 0.0)
  o_ref[...] = jnp.dot(h, w_ref[...], preferred_element_type=F32) * dinv


def _pool_body(agg_ref, zs_ref, degp_ref, b_ref, batch_ref, wfc_ref, bfc_ref,
               o_ref, pacc, cacc, *, nblk, g):
  i = pl.program_id(0)
  dinv = _dinv_blk(degp_ref, i)[:, None]
  h = (agg_ref[0] + agg_ref[1] + zs_ref[...]) * dinv + b_ref[...]
  h = jnp.maximum(h, 0.0)
  bt = batch_ref[0, 0, :]
  mask = (bt[:, None] == lax.broadcasted_iota(jnp.int32, (BLK, g), 1))
  mask = mask.astype(F32)

  @pl.when(i == 0)
  def _():
    pacc[...] = jnp.zeros_like(pacc)
    cacc[...] = jnp.zeros_like(cacc)

  pacc[...] += lax.dot_general(mask, h, (((0,), (0,)), ((), ())),
                               preferred_element_type=F32)
  cacc[...] += jnp.sum(mask, axis=0)[None, :]

  @pl.when(i == nblk - 1)
  def _():
    cnt = jnp.maximum(cacc[...], 1.0).reshape(g, 1)
    pooled = pacc[...] / cnt
    o_ref[...] = (jnp.dot(pooled, wfc_ref[...], preferred_element_type=F32)
                  + bfc_ref[...])


def _tc_mm_scale(x_pad, w, degp, npad):
  nblk = npad // BLK
  return pl.pallas_call(
      _mm_scale_body,
      grid=(nblk,),
      in_specs=[
          pl.BlockSpec((BLK, 128), lambda i: (i, 0)),
          pl.BlockSpec((128, 128), lambda i: (0, 0)),
          pl.BlockSpec((NC, npad), lambda i: (0, 0)),
      ],
      out_specs=pl.BlockSpec((BLK, 128), lambda i: (i, 0)),
      out_shape=jax.ShapeDtypeStruct((npad, 128), F32),
  )(x_pad, w, degp)


def _tc_mid(agg, zs, degp, b, w, npad):
  nblk = npad // BLK
  return pl.pallas_call(
      _mid_body,
      grid=(nblk,),
      in_specs=[
          pl.BlockSpec((NC, BLK, 128), lambda i: (0, i, 0)),
          pl.BlockSpec((BLK, 128), lambda i: (i, 0)),
          pl.BlockSpec((NC, npad), lambda i: (0, 0)),
          pl.BlockSpec((1, 128), lambda i: (0, 0)),
          pl.BlockSpec((128, 128), lambda i: (0, 0)),
      ],
      out_specs=pl.BlockSpec((BLK, 128), lambda i: (i, 0)),
      out_shape=jax.ShapeDtypeStruct((npad, 128), F32),
  )(agg, zs, degp, b.reshape(1, 128), w)


def _tc_pool(agg, zs, degp, b, batch3, wfc, bfc, npad, g, c):
  nblk = npad // BLK
  return pl.pallas_call(
      functools.partial(_pool_body, nblk=nblk, g=g),
      grid=(nblk,),
      in_specs=[
          pl.BlockSpec((NC, BLK, 128), lambda i: (0, i, 0)),
          pl.BlockSpec((BLK, 128), lambda i: (i, 0)),
          pl.BlockSpec((NC, npad), lambda i: (0, 0)),
          pl.BlockSpec((1, 128), lambda i: (0, 0)),
          pl.BlockSpec((1, 1, BLK), lambda i: (i, 0, 0)),
          pl.BlockSpec((128, c), lambda i: (0, 0)),
          pl.BlockSpec((1, c), lambda i: (0, 0)),
      ],
      out_specs=pl.BlockSpec((g, c), lambda i: (0, 0)),
      out_shape=jax.ShapeDtypeStruct((g, c), F32),
      scratch_shapes=[
          pltpu.VMEM((g, 128), F32),
          pltpu.VMEM((1, g), F32),
      ],
  )(agg, zs, degp, b.reshape(1, 128), batch3, wfc, bfc.reshape(1, c))


def kernel(x, edge_index, batch, W1, b1, W2, b2, Wfc, bfc):
  n, d = x.shape
  e = edge_index.shape[1]
  g = 64
  c = Wfc.shape[1]

  npad = ((n + BLK) // BLK) * BLK
  ekc = NW * K * IB
  epad = -(-e // ekc) * ekc
  nch_agg = epad // (NW * K)
  nch_deg = epad // (NW * K)

  x_pad = jnp.pad(x, ((0, npad - n), (0, 0)))
  spread = n + jnp.arange(epad - e, dtype=jnp.int32) % (npad - n)
  src_pad = jnp.concatenate([edge_index[0], spread])
  dst_pad = jnp.concatenate([edge_index[1], spread])
  src3a = src_pad.reshape(NW, nch_agg, K)
  dst3a = dst_pad.reshape(NW, nch_agg, K)
  dst3d = dst_pad.reshape(NW, nch_deg, K)
  batch3 = jnp.concatenate(
      [batch, jnp.full((npad - n,), -1, jnp.int32)]).reshape(-1, 1, BLK)

  degp = _sc_deg(dst3d, npad)
  zs1 = _tc_mm_scale(x_pad, W1, degp, npad)
  agg1 = _sc_agg(zs1, src3a, dst3a, npad)
  zs2 = _tc_mid(agg1, zs1, degp, b1, W2, npad)
  agg2 = _sc_agg(zs2, src3a, dst3a, npad)
  return _tc_pool(agg2, zs2, degp, b2, batch3, Wfc, bfc, npad, g, c)


# speedup vs baseline: 3.8963x; 1.1634x over previous
import functools

import jax
import jax.numpy as jnp
from jax import lax
from jax.experimental import pallas as pl
from jax.experimental.pallas import tpu as pltpu
from jax.experimental.pallas import tpu_sc as plsc

NC = 2
NS = 16
NW = NC * NS
K = 64
HH = 64
BLK = 1024
NBUF = 4
IB = 32

F32 = jnp.float32


def _deg_body(dst_hbm, out_hbm, didx_all, ones_v, zb, acc, *, nchunks, npad):
  cid = lax.axis_index("c")
  sid = lax.axis_index("s")
  wid = cid * NS + sid
  rpt = npad // NS
  for c in range(8):
    zb[pl.ds(c * 16, 16)] = jnp.zeros((16,), F32)
  for c in range(K // 16):
    ones_v[pl.ds(c * 16, 16)] = jnp.full((16,), 1.0, F32)
  r0 = sid * rpt
  for t in range(rpt // 128):
    pltpu.sync_copy(zb, acc.at[pl.ds(r0 + t * 128, 128)])
  pltpu.sync_copy(dst_hbm.at[wid], didx_all)
  plsc.subcore_barrier()

  def body(j, carry):
    pltpu.sync_copy(ones_v, acc.at[didx_all.at[j]], add=True)
    return carry

  lax.fori_loop(0, nchunks, body, 0)
  plsc.subcore_barrier()
  for t in range(rpt // 128):
    pltpu.sync_copy(acc.at[pl.ds(r0 + t * 128, 128)],
                    out_hbm.at[cid, pl.ds(r0 + t * 128, 128)])


def _agg_body(zs_hbm, src_hbm, dst_hbm, out_hbm, sidx_blk, didx_blk,
              rows, zb, acc, *sems, nchunks, npad):
  cid = lax.axis_index("c")
  sid = lax.axis_index("s")
  rpt = npad // NS
  for i in range(16):
    for c in range(8):
      zb[i, pl.ds(c * 16, 16)] = jnp.zeros((16,), F32)
  r0 = sid * rpt
  for t in range(rpt // 16):
    pltpu.sync_copy(zb, acc.at[pl.ds(r0 + t * 16, 16)])
  wid = cid * NS + sid
  plsc.subcore_barrier()
  gsem = sems[:NBUF]
  ssem = sems[NBUF:]

  def gissue(c, b):
    pltpu.async_copy(zs_hbm.at[sidx_blk.at[c]], rows.at[b], gsem[b])

  def gwait(c, b):
    pltpu.make_async_copy(zs_hbm.at[sidx_blk.at[c]], rows.at[b],
                          gsem[b]).wait()

  def sissue(c, b):
    pltpu.async_copy(rows.at[b], acc.at[didx_blk.at[c]], ssem[b], add=True)

  def swait(c, b):
    pltpu.make_async_copy(rows.at[b], acc.at[didx_blk.at[c]],
                          ssem[b]).wait()

  def outer(ib, carry):
    pltpu.sync_copy(src_hbm.at[wid, pl.ds(ib * IB, IB)], sidx_blk)
    pltpu.sync_copy(dst_hbm.at[wid, pl.ds(ib * IB, IB)], didx_blk)
    for b in range(NBUF):
      gissue(b, b)

    def group(i, carry2):
      c0 = NBUF * i
      for b in range(NBUF):
        c = c0 + b
        gwait(c, b)
        sissue(c, b)
        swait(c, b)

        @pl.when(c + NBUF < IB)
        def _():
          gissue(c + NBUF, b)

      return carry2

    lax.fori_loop(0, IB // NBUF, group, 0)
    return carry

  lax.fori_loop(0, nchunks // IB, outer, 0)
  plsc.subcore_barrier()
  for t in range(rpt // 128):
    pltpu.sync_copy(acc.at[pl.ds(r0 + t * 128, 128)],
                    out_hbm.at[cid, pl.ds(r0 + t * 128, 128)])


def _sc_deg(dst3, npad):
  nchunks = dst3.shape[1]
  mesh = plsc.VectorSubcoreMesh(core_axis_name="c", subcore_axis_name="s")
  fn = pl.kernel(
      functools.partial(_deg_body, nchunks=nchunks, npad=npad),
      mesh=mesh,
      out_type=jax.ShapeDtypeStruct((NC, npad), F32),
      scratch_types=[
          pltpu.VMEM((nchunks, K), jnp.int32),
          pltpu.VMEM((K,), F32),
          pltpu.VMEM((128,), F32),
          pltpu.VMEM_SHARED((npad,), F32),
      ],
  )
  return fn(dst3)


def _sc_agg(zs, src3, dst3, npad):
  nchunks = src3.shape[1]
  mesh = plsc.VectorSubcoreMesh(core_axis_name="c", subcore_axis_name="s")
  fn = pl.kernel(
      functools.partial(_agg_body, nchunks=nchunks, npad=npad),
      mesh=mesh,
      out_type=jax.ShapeDtypeStruct((NC, npad, 128), F32),
      scratch_types=[
          pltpu.VMEM((IB, K), jnp.int32),
          pltpu.VMEM((IB, K), jnp.int32),
          pltpu.VMEM((NBUF, K, 128), F32),
          pltpu.VMEM((16, 128), F32),
          pltpu.VMEM_SHARED((npad, 128), F32),
      ] + [pltpu.SemaphoreType.DMA] * (2 * NBUF),
  )
  return fn(zs, src3, dst3)


def _dinv_blk(degp_ref, i):
  d = degp_ref[:, pl.ds(i * BLK, BLK)]
  return lax.rsqrt(d[0] + d[1] + 1.0)


def _split(z):
  return z


def _mm_scale_body(x_ref, w_ref, degp_ref, o_ref):
  i = pl.program_id(0)
  dinv = _dinv_blk(degp_ref, i)
  z = jnp.dot(x_ref[...], w_ref[...], preferred_element_type=F32)
  o_ref[...] = z * dinv[:, None]


def _mid_body(agg_ref, zs_ref, degp_ref, b_ref, w_ref, o_ref):
  i = pl.program_id(0)
  dinv = _dinv_blk(degp_ref, i)[:, None]
  h = (agg_ref[0] + agg_ref[1] + zs_ref[...]) * dinv + b_ref[...]
  h = jnp.maximum(h, 0.0)
  o_ref[...] = jnp.dot(h, w_ref[...], preferred_element_type=F32) * dinv


def _pool_body(agg_ref, zs_ref, degp_ref, b_ref, batch_ref, wfc_ref, bfc_ref,
               o_ref, pacc, cacc, *, nblk, g):
  i = pl.program_id(0)
  dinv = _dinv_blk(degp_ref, i)[:, None]
  h = (agg_ref[0] + agg_ref[1] + zs_ref[...]) * dinv + b_ref[...]
  h = jnp.maximum(h, 0.0)
  bt = batch_ref[0, 0, :]
  mask = (bt[:, None] == lax.broadcasted_iota(jnp.int32, (BLK, g), 1))
  mask = mask.astype(F32)

  @pl.when(i == 0)
  def _():
    pacc[...] = jnp.zeros_like(pacc)
    cacc[...] = jnp.zeros_like(cacc)

  pacc[...] += lax.dot_general(mask, h, (((0,), (0,)), ((), ())),
                               preferred_element_type=F32)
  cacc[...] += jnp.sum(mask, axis=0)[None, :]

  @pl.when(i == nblk - 1)
  def _():
    cnt = jnp.maximum(cacc[...], 1.0).reshape(g, 1)
    pooled = pacc[...] / cnt
    o_ref[...] = (jnp.dot(pooled, wfc_ref[...], preferred_element_type=F32)
                  + bfc_ref[...])


def _tc_mm_scale(x_pad, w, degp, npad):
  nblk = npad // BLK
  return pl.pallas_call(
      _mm_scale_body,
      grid=(nblk,),
      in_specs=[
          pl.BlockSpec((BLK, 128), lambda i: (i, 0)),
          pl.BlockSpec((128, 128), lambda i: (0, 0)),
          pl.BlockSpec((NC, npad), lambda i: (0, 0)),
      ],
      out_specs=pl.BlockSpec((BLK, 128), lambda i: (i, 0)),
      out_shape=jax.ShapeDtypeStruct((npad, 128), F32),
  )(x_pad, w, degp)


def _tc_mid(agg, zs, degp, b, w, npad):
  nblk = npad // BLK
  return pl.pallas_call(
      _mid_body,
      grid=(nblk,),
      in_specs=[
          pl.BlockSpec((NC, BLK, 128), lambda i: (0, i, 0)),
          pl.BlockSpec((BLK, 128), lambda i: (i, 0)),
          pl.BlockSpec((NC, npad), lambda i: (0, 0)),
          pl.BlockSpec((1, 128), lambda i: (0, 0)),
          pl.BlockSpec((128, 128), lambda i: (0, 0)),
      ],
      out_specs=pl.BlockSpec((BLK, 128), lambda i: (i, 0)),
      out_shape=jax.ShapeDtypeStruct((npad, 128), F32),
  )(agg, zs, degp, b.reshape(1, 128), w)


def _tc_pool(agg, zs, degp, b, batch3, wfc, bfc, npad, g, c):
  nblk = npad // BLK
  return pl.pallas_call(
      functools.partial(_pool_body, nblk=nblk, g=g),
      grid=(nblk,),
      in_specs=[
          pl.BlockSpec((NC, BLK, 128), lambda i: (0, i, 0)),
          pl.BlockSpec((BLK, 128), lambda i: (i, 0)),
          pl.BlockSpec((NC, npad), lambda i: (0, 0)),
          pl.BlockSpec((1, 128), lambda i: (0, 0)),
          pl.BlockSpec((1, 1, BLK), lambda i: (i, 0, 0)),
          pl.BlockSpec((128, c), lambda i: (0, 0)),
          pl.BlockSpec((1, c), lambda i: (0, 0)),
      ],
      out_specs=pl.BlockSpec((g, c), lambda i: (0, 0)),
      out_shape=jax.ShapeDtypeStruct((g, c), F32),
      scratch_shapes=[
          pltpu.VMEM((g, 128), F32),
          pltpu.VMEM((1, g), F32),
      ],
  )(agg, zs, degp, b.reshape(1, 128), batch3, wfc, bfc.reshape(1, c))


def kernel(x, edge_index, batch, W1, b1, W2, b2, Wfc, bfc):
  n, d = x.shape
  e = edge_index.shape[1]
  g = 64
  c = Wfc.shape[1]

  npad = ((n + BLK) // BLK) * BLK
  ekc = NW * K * IB
  epad = -(-e // ekc) * ekc
  nch_agg = epad // (NW * K)
  nch_deg = epad // (NW * K)

  x_pad = jnp.pad(x, ((0, npad - n), (0, 0)))
  spread = n + jnp.arange(epad - e, dtype=jnp.int32) % (npad - n)
  src_pad = jnp.concatenate([edge_index[0], spread])
  dst_pad = jnp.concatenate([edge_index[1], spread])
  src3a = src_pad.reshape(NW, nch_agg, K)
  dst3a = dst_pad.reshape(NW, nch_agg, K)
  dst3d = dst_pad.reshape(NW, nch_deg, K)
  batch3 = jnp.concatenate(
      [batch, jnp.full((npad - n,), -1, jnp.int32)]).reshape(-1, 1, BLK)

  degp = _sc_deg(dst3d, npad)
  zs1 = _tc_mm_scale(x_pad, W1, degp, npad)
  agg1 = _sc_agg(zs1, src3a, dst3a, npad)
  zs2 = _tc_mid(agg1, zs1, degp, b1, W2, npad)
  agg2 = _sc_agg(zs2, src3a, dst3a, npad)
  return _tc_pool(agg2, zs2, degp, b2, batch3, Wfc, bfc, npad, g, c)


# speedup vs baseline: 3.9798x; 1.0214x over previous
import functools

import jax
import jax.numpy as jnp
from jax import lax
from jax.experimental import pallas as pl
from jax.experimental.pallas import tpu as pltpu
from jax.experimental.pallas import tpu_sc as plsc

NC = 2
NS = 16
NW = NC * NS
K = 64
HH = 64
BLK = 2048
NBUF = 4
IB = 32

F32 = jnp.float32


def _deg_body(dst_hbm, out_hbm, didx_all, ones_v, zb, acc, *, nchunks, npad):
  cid = lax.axis_index("c")
  sid = lax.axis_index("s")
  wid = cid * NS + sid
  rpt = npad // NS
  for c in range(8):
    zb[pl.ds(c * 16, 16)] = jnp.zeros((16,), F32)
  for c in range(K // 16):
    ones_v[pl.ds(c * 16, 16)] = jnp.full((16,), 1.0, F32)
  r0 = sid * rpt
  for t in range(rpt // 128):
    pltpu.sync_copy(zb, acc.at[pl.ds(r0 + t * 128, 128)])
  pltpu.sync_copy(dst_hbm.at[wid], didx_all)
  plsc.subcore_barrier()

  def body(j, carry):
    pltpu.sync_copy(ones_v, acc.at[didx_all.at[j]], add=True)
    return carry

  lax.fori_loop(0, nchunks, body, 0)
  plsc.subcore_barrier()
  for t in range(rpt // 128):
    pltpu.sync_copy(acc.at[pl.ds(r0 + t * 128, 128)],
                    out_hbm.at[cid, pl.ds(r0 + t * 128, 128)])


def _agg_body(zs_hbm, src_hbm, dst_hbm, out_hbm, sidx_blk, didx_blk,
              rows, zb, acc, *sems, nchunks, npad):
  cid = lax.axis_index("c")
  sid = lax.axis_index("s")
  rpt = npad // NS
  for i in range(16):
    for c in range(8):
      zb[i, pl.ds(c * 16, 16)] = jnp.zeros((16,), F32)
  r0 = sid * rpt
  for t in range(rpt // 16):
    pltpu.sync_copy(zb, acc.at[pl.ds(r0 + t * 16, 16)])
  wid = cid * NS + sid
  plsc.subcore_barrier()
  gsem = sems[:NBUF]
  ssem = sems[NBUF:]

  def gissue(c, b):
    pltpu.async_copy(zs_hbm.at[sidx_blk.at[c]], rows.at[b], gsem[b])

  def gwait(c, b):
    pltpu.make_async_copy(zs_hbm.at[sidx_blk.at[c]], rows.at[b],
                          gsem[b]).wait()

  def sissue(c, b):
    pltpu.async_copy(rows.at[b], acc.at[didx_blk.at[c]], ssem[b], add=True)

  def swait(c, b):
    pltpu.make_async_copy(rows.at[b], acc.at[didx_blk.at[c]],
                          ssem[b]).wait()

  def outer(ib, carry):
    pltpu.sync_copy(src_hbm.at[wid, pl.ds(ib * IB, IB)], sidx_blk)
    pltpu.sync_copy(dst_hbm.at[wid, pl.ds(ib * IB, IB)], didx_blk)
    for b in range(NBUF):
      gissue(b, b)

    def group(i, carry2):
      c0 = NBUF * i
      for b in range(NBUF):
        c = c0 + b
        gwait(c, b)
        sissue(c, b)
        swait(c, b)

        @pl.when(c + NBUF < IB)
        def _():
          gissue(c + NBUF, b)

      return carry2

    lax.fori_loop(0, IB // NBUF, group, 0)
    return carry

  lax.fori_loop(0, nchunks // IB, outer, 0)
  plsc.subcore_barrier()
  for t in range(rpt // 128):
    pltpu.sync_copy(acc.at[pl.ds(r0 + t * 128, 128)],
                    out_hbm.at[cid, pl.ds(r0 + t * 128, 128)])


def _sc_deg(dst3, npad):
  nchunks = dst3.shape[1]
  mesh = plsc.VectorSubcoreMesh(core_axis_name="c", subcore_axis_name="s")
  fn = pl.kernel(
      functools.partial(_deg_body, nchunks=nchunks, npad=npad),
      mesh=mesh,
      out_type=jax.ShapeDtypeStruct((NC, npad), F32),
      scratch_types=[
          pltpu.VMEM((nchunks, K), jnp.int32),
          pltpu.VMEM((K,), F32),
          pltpu.VMEM((128,), F32),
          pltpu.VMEM_SHARED((npad,), F32),
      ],
  )
  return fn(dst3)


def _sc_agg(zs, src3, dst3, npad):
  nchunks = src3.shape[1]
  mesh = plsc.VectorSubcoreMesh(core_axis_name="c", subcore_axis_name="s")
  fn = pl.kernel(
      functools.partial(_agg_body, nchunks=nchunks, npad=npad),
      mesh=mesh,
      out_type=jax.ShapeDtypeStruct((NC, npad, 128), F32),
      scratch_types=[
          pltpu.VMEM((IB, K), jnp.int32),
          pltpu.VMEM((IB, K), jnp.int32),
          pltpu.VMEM((NBUF, K, 128), F32),
          pltpu.VMEM((16, 128), F32),
          pltpu.VMEM_SHARED((npad, 128), F32),
      ] + [pltpu.SemaphoreType.DMA] * (2 * NBUF),
  )
  return fn(zs, src3, dst3)


def _dinv_blk(degp_ref, i):
  d = degp_ref[:, pl.ds(i * BLK, BLK)]
  return lax.rsqrt(d[0] + d[1] + 1.0)


def _split(z):
  return z


def _mm_scale_body(x_ref, w_ref, degp_ref, o_ref):
  i = pl.program_id(0)
  dinv = _dinv_blk(degp_ref, i)
  z = jnp.dot(x_ref[...], w_ref[...], preferred_element_type=F32)
  o_ref[...] = z * dinv[:, None]


def _mid_body(agg_ref, zs_ref, degp_ref, b_ref, w_ref, o_ref):
  i = pl.program_id(0)
  dinv = _dinv_blk(degp_ref, i)[:, None]
  h = (agg_ref[0] + agg_ref[1] + zs_ref[...]) * dinv + b_ref[...]
  h = jnp.maximum(h, 0.0)
  o_ref[...] = jnp.dot(h, w_ref[...], preferred_element_type=F32) * dinv


def _pool_body(agg_ref, zs_ref, degp_ref, b_ref, batch_ref, wfc_ref, bfc_ref,
               o_ref, pacc, cacc, *, nblk, g):
  i = pl.program_id(0)
  dinv = _dinv_blk(degp_ref, i)[:, None]
  h = (agg_ref[0] + agg_ref[1] + zs_ref[...]) * dinv + b_ref[...]
  h = jnp.maximum(h, 0.0)
  bt = batch_ref[0, 0, :]
  mask = (bt[:, None] == lax.broadcasted_iota(jnp.int32, (BLK, g), 1))
  mask = mask.astype(F32)

  @pl.when(i == 0)
  def _():
    pacc[...] = jnp.zeros_like(pacc)
    cacc[...] = jnp.zeros_like(cacc)

  pacc[...] += lax.dot_general(mask, h, (((0,), (0,)), ((), ())),
                               preferred_element_type=F32)
  cacc[...] += jnp.sum(mask, axis=0)[None, :]

  @pl.when(i == nblk - 1)
  def _():
    cnt = jnp.maximum(cacc[...], 1.0).reshape(g, 1)
    pooled = pacc[...] / cnt
    o_ref[...] = (jnp.dot(pooled, wfc_ref[...], preferred_element_type=F32)
                  + bfc_ref[...])


def _tc_mm_scale(x_pad, w, degp, npad):
  nblk = npad // BLK
  return pl.pallas_call(
      _mm_scale_body,
      grid=(nblk,),
      in_specs=[
          pl.BlockSpec((BLK, 128), lambda i: (i, 0)),
          pl.BlockSpec((128, 128), lambda i: (0, 0)),
          pl.BlockSpec((NC, npad), lambda i: (0, 0)),
      ],
      out_specs=pl.BlockSpec((BLK, 128), lambda i: (i, 0)),
      out_shape=jax.ShapeDtypeStruct((npad, 128), F32),
  )(x_pad, w, degp)


def _tc_mid(agg, zs, degp, b, w, npad):
  nblk = npad // BLK
  return pl.pallas_call(
      _mid_body,
      grid=(nblk,),
      in_specs=[
          pl.BlockSpec((NC, BLK, 128), lambda i: (0, i, 0)),
          pl.BlockSpec((BLK, 128), lambda i: (i, 0)),
          pl.BlockSpec((NC, npad), lambda i: (0, 0)),
          pl.BlockSpec((1, 128), lambda i: (0, 0)),
          pl.BlockSpec((128, 128), lambda i: (0, 0)),
      ],
      out_specs=pl.BlockSpec((BLK, 128), lambda i: (i, 0)),
      out_shape=jax.ShapeDtypeStruct((npad, 128), F32),
  )(agg, zs, degp, b.reshape(1, 128), w)


def _tc_pool(agg, zs, degp, b, batch3, wfc, bfc, npad, g, c):
  nblk = npad // BLK
  return pl.pallas_call(
      functools.partial(_pool_body, nblk=nblk, g=g),
      grid=(nblk,),
      in_specs=[
          pl.BlockSpec((NC, BLK, 128), lambda i: (0, i, 0)),
          pl.BlockSpec((BLK, 128), lambda i: (i, 0)),
          pl.BlockSpec((NC, npad), lambda i: (0, 0)),
          pl.BlockSpec((1, 128), lambda i: (0, 0)),
          pl.BlockSpec((1, 1, BLK), lambda i: (i, 0, 0)),
          pl.BlockSpec((128, c), lambda i: (0, 0)),
          pl.BlockSpec((1, c), lambda i: (0, 0)),
      ],
      out_specs=pl.BlockSpec((g, c), lambda i: (0, 0)),
      out_shape=jax.ShapeDtypeStruct((g, c), F32),
      scratch_shapes=[
          pltpu.VMEM((g, 128), F32),
          pltpu.VMEM((1, g), F32),
      ],
  )(agg, zs, degp, b.reshape(1, 128), batch3, wfc, bfc.reshape(1, c))


def kernel(x, edge_index, batch, W1, b1, W2, b2, Wfc, bfc):
  n, d = x.shape
  e = edge_index.shape[1]
  g = 64
  c = Wfc.shape[1]

  npad = ((n + BLK) // BLK) * BLK
  ekc = NW * K * IB
  epad = -(-e // ekc) * ekc
  nch_agg = epad // (NW * K)
  nch_deg = epad // (NW * K)

  x_pad = jnp.pad(x, ((0, npad - n), (0, 0)))
  spread = n + jnp.arange(epad - e, dtype=jnp.int32) % (npad - n)
  src_pad = jnp.concatenate([edge_index[0], spread])
  dst_pad = jnp.concatenate([edge_index[1], spread])
  src3a = src_pad.reshape(NW, nch_agg, K)
  dst3a = dst_pad.reshape(NW, nch_agg, K)
  dst3d = dst_pad.reshape(NW, nch_deg, K)
  batch3 = jnp.concatenate(
      [batch, jnp.full((npad - n,), -1, jnp.int32)]).reshape(-1, 1, BLK)

  degp = _sc_deg(dst3d, npad)
  zs1 = _tc_mm_scale(x_pad, W1, degp, npad)
  agg1 = _sc_agg(zs1, src3a, dst3a, npad)
  zs2 = _tc_mid(agg1, zs1, degp, b1, W2, npad)
  agg2 = _sc_agg(zs2, src3a, dst3a, npad)
  return _tc_pool(agg2, zs2, degp, b2, batch3, Wfc, bfc, npad, g, c)


# speedup vs baseline: 3.9903x; 1.0026x over previous
import functools

import jax
import jax.numpy as jnp
from jax import lax
from jax.experimental import pallas as pl
from jax.experimental.pallas import tpu as pltpu
from jax.experimental.pallas import tpu_sc as plsc

NC = 2
NS = 16
NW = NC * NS
K = 64
HH = 64
BLK = 10240
NBUF = 4
IB = 32

F32 = jnp.float32


def _deg_body(dst_hbm, out_hbm, didx_all, ones_v, zb, acc, *, nchunks, npad):
  cid = lax.axis_index("c")
  sid = lax.axis_index("s")
  wid = cid * NS + sid
  rpt = npad // NS
  for c in range(8):
    zb[pl.ds(c * 16, 16)] = jnp.zeros((16,), F32)
  for c in range(K // 16):
    ones_v[pl.ds(c * 16, 16)] = jnp.full((16,), 1.0, F32)
  r0 = sid * rpt
  for t in range(rpt // 128):
    pltpu.sync_copy(zb, acc.at[pl.ds(r0 + t * 128, 128)])
  pltpu.sync_copy(dst_hbm.at[wid], didx_all)
  plsc.subcore_barrier()

  def body(j, carry):
    pltpu.sync_copy(ones_v, acc.at[didx_all.at[j]], add=True)
    return carry

  lax.fori_loop(0, nchunks, body, 0)
  plsc.subcore_barrier()
  for t in range(rpt // 128):
    pltpu.sync_copy(acc.at[pl.ds(r0 + t * 128, 128)],
                    out_hbm.at[cid, pl.ds(r0 + t * 128, 128)])


def _agg_body(zs_hbm, src_hbm, dst_hbm, out_hbm, sidx_blk, didx_blk,
              rows, zb, acc, *sems, nchunks, npad):
  cid = lax.axis_index("c")
  sid = lax.axis_index("s")
  rpt = npad // NS
  for i in range(16):
    for c in range(8):
      zb[i, pl.ds(c * 16, 16)] = jnp.zeros((16,), F32)
  r0 = sid * rpt
  for t in range(rpt // 16):
    pltpu.sync_copy(zb, acc.at[pl.ds(r0 + t * 16, 16)])
  wid = cid * NS + sid
  plsc.subcore_barrier()
  gsem = sems[:NBUF]
  ssem = sems[NBUF:]

  def gissue(c, b):
    pltpu.async_copy(zs_hbm.at[sidx_blk.at[c]], rows.at[b], gsem[b])

  def gwait(c, b):
    pltpu.make_async_copy(zs_hbm.at[sidx_blk.at[c]], rows.at[b],
                          gsem[b]).wait()

  def sissue(c, b):
    pltpu.async_copy(rows.at[b], acc.at[didx_blk.at[c]], ssem[b], add=True)

  def swait(c, b):
    pltpu.make_async_copy(rows.at[b], acc.at[didx_blk.at[c]],
                          ssem[b]).wait()

  def outer(ib, carry):
    pltpu.sync_copy(src_hbm.at[wid, pl.ds(ib * IB, IB)], sidx_blk)
    pltpu.sync_copy(dst_hbm.at[wid, pl.ds(ib * IB, IB)], didx_blk)
    for b in range(NBUF):
      gissue(b, b)

    def group(i, carry2):
      c0 = NBUF * i
      for b in range(NBUF):
        c = c0 + b
        gwait(c, b)
        sissue(c, b)
        swait(c, b)

        @pl.when(c + NBUF < IB)
        def _():
          gissue(c + NBUF, b)

      return carry2

    lax.fori_loop(0, IB // NBUF, group, 0)
    return carry

  lax.fori_loop(0, nchunks // IB, outer, 0)
  plsc.subcore_barrier()
  for t in range(rpt // 128):
    pltpu.sync_copy(acc.at[pl.ds(r0 + t * 128, 128)],
                    out_hbm.at[cid, pl.ds(r0 + t * 128, 128)])


def _sc_deg(dst3, npad):
  nchunks = dst3.shape[1]
  mesh = plsc.VectorSubcoreMesh(core_axis_name="c", subcore_axis_name="s")
  fn = pl.kernel(
      functools.partial(_deg_body, nchunks=nchunks, npad=npad),
      mesh=mesh,
      out_type=jax.ShapeDtypeStruct((NC, npad), F32),
      scratch_types=[
          pltpu.VMEM((nchunks, K), jnp.int32),
          pltpu.VMEM((K,), F32),
          pltpu.VMEM((128,), F32),
          pltpu.VMEM_SHARED((npad,), F32),
      ],
  )
  return fn(dst3)


def _sc_agg(zs, src3, dst3, npad):
  nchunks = src3.shape[1]
  mesh = plsc.VectorSubcoreMesh(core_axis_name="c", subcore_axis_name="s")
  fn = pl.kernel(
      functools.partial(_agg_body, nchunks=nchunks, npad=npad),
      mesh=mesh,
      out_type=jax.ShapeDtypeStruct((NC, npad, 128), F32),
      scratch_types=[
          pltpu.VMEM((IB, K), jnp.int32),
          pltpu.VMEM((IB, K), jnp.int32),
          pltpu.VMEM((NBUF, K, 128), F32),
          pltpu.VMEM((16, 128), F32),
          pltpu.VMEM_SHARED((npad, 128), F32),
      ] + [pltpu.SemaphoreType.DMA] * (2 * NBUF),
  )
  return fn(zs, src3, dst3)


def _dinv_blk(degp_ref, i):
  d = degp_ref[:, pl.ds(i * BLK, BLK)]
  return lax.rsqrt(d[0] + d[1] + 1.0)


def _split(z):
  return z


def _mm_scale_body(x_ref, w_ref, degp_ref, o_ref):
  i = pl.program_id(0)
  dinv = _dinv_blk(degp_ref, i)
  z = jnp.dot(x_ref[...], w_ref[...], preferred_element_type=F32)
  o_ref[...] = z * dinv[:, None]


def _mid_body(agg_ref, zs_ref, degp_ref, b_ref, w_ref, o_ref):
  i = pl.program_id(0)
  dinv = _dinv_blk(degp_ref, i)[:, None]
  h = (agg_ref[0] + agg_ref[1] + zs_ref[...]) * dinv + b_ref[...]
  h = jnp.maximum(h, 0.0)
  o_ref[...] = jnp.dot(h, w_ref[...], preferred_element_type=F32) * dinv


def _pool_body(agg_ref, zs_ref, degp_ref, b_ref, batch_ref, wfc_ref, bfc_ref,
               o_ref, pacc, cacc, *, nblk, g):
  i = pl.program_id(0)
  dinv = _dinv_blk(degp_ref, i)[:, None]
  h = (agg_ref[0] + agg_ref[1] + zs_ref[...]) * dinv + b_ref[...]
  h = jnp.maximum(h, 0.0)
  bt = batch_ref[0, 0, :]
  mask = (bt[:, None] == lax.broadcasted_iota(jnp.int32, (BLK, g), 1))
  mask = mask.astype(F32)

  @pl.when(i == 0)
  def _():
    pacc[...] = jnp.zeros_like(pacc)
    cacc[...] = jnp.zeros_like(cacc)

  pacc[...] += lax.dot_general(mask, h, (((0,), (0,)), ((), ())),
                               preferred_element_type=F32)
  cacc[...] += jnp.sum(mask, axis=0)[None, :]

  @pl.when(i == nblk - 1)
  def _():
    cnt = jnp.maximum(cacc[...], 1.0).reshape(g, 1)
    pooled = pacc[...] / cnt
    o_ref[...] = (jnp.dot(pooled, wfc_ref[...], preferred_element_type=F32)
                  + bfc_ref[...])


def _tc_mm_scale(x_pad, w, degp, npad):
  nblk = npad // BLK
  return pl.pallas_call(
      _mm_scale_body,
      grid=(nblk,),
      in_specs=[
          pl.BlockSpec((BLK, 128), lambda i: (i, 0)),
          pl.BlockSpec((128, 128), lambda i: (0, 0)),
          pl.BlockSpec((NC, npad), lambda i: (0, 0)),
      ],
      out_specs=pl.BlockSpec((BLK, 128), lambda i: (i, 0)),
      out_shape=jax.ShapeDtypeStruct((npad, 128), F32),
  )(x_pad, w, degp)


def _tc_mid(agg, zs, degp, b, w, npad):
  nblk = npad // BLK
  return pl.pallas_call(
      _mid_body,
      grid=(nblk,),
      in_specs=[
          pl.BlockSpec((NC, BLK, 128), lambda i: (0, i, 0)),
          pl.BlockSpec((BLK, 128), lambda i: (i, 0)),
          pl.BlockSpec((NC, npad), lambda i: (0, 0)),
          pl.BlockSpec((1, 128), lambda i: (0, 0)),
          pl.BlockSpec((128, 128), lambda i: (0, 0)),
      ],
      out_specs=pl.BlockSpec((BLK, 128), lambda i: (i, 0)),
      out_shape=jax.ShapeDtypeStruct((npad, 128), F32),
  )(agg, zs, degp, b.reshape(1, 128), w)


def _tc_pool(agg, zs, degp, b, batch3, wfc, bfc, npad, g, c):
  nblk = npad // BLK
  return pl.pallas_call(
      functools.partial(_pool_body, nblk=nblk, g=g),
      grid=(nblk,),
      in_specs=[
          pl.BlockSpec((NC, BLK, 128), lambda i: (0, i, 0)),
          pl.BlockSpec((BLK, 128), lambda i: (i, 0)),
          pl.BlockSpec((NC, npad), lambda i: (0, 0)),
          pl.BlockSpec((1, 128), lambda i: (0, 0)),
          pl.BlockSpec((1, 1, BLK), lambda i: (i, 0, 0)),
          pl.BlockSpec((128, c), lambda i: (0, 0)),
          pl.BlockSpec((1, c), lambda i: (0, 0)),
      ],
      out_specs=pl.BlockSpec((g, c), lambda i: (0, 0)),
      out_shape=jax.ShapeDtypeStruct((g, c), F32),
      scratch_shapes=[
          pltpu.VMEM((g, 128), F32),
          pltpu.VMEM((1, g), F32),
      ],
  )(agg, zs, degp, b.reshape(1, 128), batch3, wfc, bfc.reshape(1, c))


def kernel(x, edge_index, batch, W1, b1, W2, b2, Wfc, bfc):
  n, d = x.shape
  e = edge_index.shape[1]
  g = 64
  c = Wfc.shape[1]

  npad = ((n + BLK) // BLK) * BLK
  ekc = NW * K * IB
  epad = -(-e // ekc) * ekc
  nch_agg = epad // (NW * K)
  nch_deg = epad // (NW * K)

  x_pad = jnp.pad(x, ((0, npad - n), (0, 0)))
  spread = n + jnp.arange(epad - e, dtype=jnp.int32) % (npad - n)
  src_pad = jnp.concatenate([edge_index[0], spread])
  dst_pad = jnp.concatenate([edge_index[1], spread])
  src3a = src_pad.reshape(NW, nch_agg, K)
  dst3a = dst_pad.reshape(NW, nch_agg, K)
  dst3d = dst_pad.reshape(NW, nch_deg, K)
  batch3 = jnp.concatenate(
      [batch, jnp.full((npad - n,), -1, jnp.int32)]).reshape(-1, 1, BLK)

  degp = _sc_deg(dst3d, npad)
  zs1 = _tc_mm_scale(x_pad, W1, degp, npad)
  agg1 = _sc_agg(zs1, src3a, dst3a, npad)
  zs2 = _tc_mid(agg1, zs1, degp, b1, W2, npad)
  agg2 = _sc_agg(zs2, src3a, dst3a, npad)
  return _tc_pool(agg2, zs2, degp, b2, batch3, Wfc, bfc, npad, g, c)


# speedup vs baseline: 3.9990x; 1.0022x over previous
import functools

import jax
import jax.numpy as jnp
from jax import lax
from jax.experimental import pallas as pl
from jax.experimental.pallas import tpu as pltpu
from jax.experimental.pallas import tpu_sc as plsc

NC = 2
NS = 16
NW = NC * NS
K = 64
BLK = 10240
NBUF = 4
IB = 32

F32 = jnp.float32


def _deg_body(dst_hbm, out_hbm, didx_all, ones_v, zb, acc, *, nchunks, npad):
  cid = lax.axis_index("c")
  sid = lax.axis_index("s")
  wid = cid * NS + sid
  rpt = npad // NS
  for c in range(8):
    zb[pl.ds(c * 16, 16)] = jnp.zeros((16,), F32)
  for c in range(K // 16):
    ones_v[pl.ds(c * 16, 16)] = jnp.full((16,), 1.0, F32)
  r0 = sid * rpt
  for t in range(rpt // 128):
    pltpu.sync_copy(zb, acc.at[pl.ds(r0 + t * 128, 128)])
  pltpu.sync_copy(dst_hbm.at[wid], didx_all)
  plsc.subcore_barrier()

  def body(j, carry):
    pltpu.sync_copy(ones_v, acc.at[didx_all.at[j]], add=True)
    return carry

  lax.fori_loop(0, nchunks, body, 0)
  plsc.subcore_barrier()
  for t in range(rpt // 128):
    pltpu.sync_copy(acc.at[pl.ds(r0 + t * 128, 128)],
                    out_hbm.at[cid, pl.ds(r0 + t * 128, 128)])


def _agg_body(zs_hbm, src_hbm, dst_hbm, out_hbm, sidx_blk, didx_blk,
              rows, zb, acc, *sems, nchunks, npad):
  cid = lax.axis_index("c")
  sid = lax.axis_index("s")
  rpt = npad // NS
  for i in range(16):
    for c in range(8):
      zb[i, pl.ds(c * 16, 16)] = jnp.zeros((16,), F32)
  r0 = sid * rpt
  for t in range(rpt // 16):
    pltpu.sync_copy(zb, acc.at[pl.ds(r0 + t * 16, 16)])
  wid = cid * NS + sid
  plsc.subcore_barrier()
  gsem = sems[:NBUF]
  ssem = sems[NBUF:]

  def gissue(c, b):
    pltpu.async_copy(zs_hbm.at[sidx_blk.at[c]], rows.at[b], gsem[b])

  def gwait(c, b):
    pltpu.make_async_copy(zs_hbm.at[sidx_blk.at[c]], rows.at[b],
                          gsem[b]).wait()

  def sissue(c, b):
    pltpu.async_copy(rows.at[b], acc.at[didx_blk.at[c]], ssem[b], add=True)

  def swait(c, b):
    pltpu.make_async_copy(rows.at[b], acc.at[didx_blk.at[c]],
                          ssem[b]).wait()

  def outer(ib, carry):
    pltpu.sync_copy(src_hbm.at[wid, pl.ds(ib * IB, IB)], sidx_blk)
    pltpu.sync_copy(dst_hbm.at[wid, pl.ds(ib * IB, IB)], didx_blk)
    for b in range(NBUF):
      gissue(b, b)

    def group(i, carry2):
      c0 = NBUF * i
      for b in range(NBUF):
        c = c0 + b
        gwait(c, b)
        sissue(c, b)
        swait(c, b)

        @pl.when(c + NBUF < IB)
        def _():
          gissue(c + NBUF, b)

      return carry2

    lax.fori_loop(0, IB // NBUF, group, 0)
    return carry

  lax.fori_loop(0, nchunks // IB, outer, 0)
  plsc.subcore_barrier()
  for t in range(rpt // 128):
    pltpu.sync_copy(acc.at[pl.ds(r0 + t * 128, 128)],
                    out_hbm.at[cid, pl.ds(r0 + t * 128, 128)])


def _sc_deg(dst3, npad):
  nchunks = dst3.shape[1]
  mesh = plsc.VectorSubcoreMesh(core_axis_name="c", subcore_axis_name="s")
  fn = pl.kernel(
      functools.partial(_deg_body, nchunks=nchunks, npad=npad),
      mesh=mesh,
      out_type=jax.ShapeDtypeStruct((NC, npad), F32),
      scratch_types=[
          pltpu.VMEM((nchunks, K), jnp.int32),
          pltpu.VMEM((K,), F32),
          pltpu.VMEM((128,), F32),
          pltpu.VMEM_SHARED((npad,), F32),
      ],
  )
  return fn(dst3)


def _sc_agg(zs, src3, dst3, npad):
  nchunks = src3.shape[1]
  mesh = plsc.VectorSubcoreMesh(core_axis_name="c", subcore_axis_name="s")
  fn = pl.kernel(
      functools.partial(_agg_body, nchunks=nchunks, npad=npad),
      mesh=mesh,
      out_type=jax.ShapeDtypeStruct((NC, npad, 128), F32),
      scratch_types=[
          pltpu.VMEM((IB, K), jnp.int32),
          pltpu.VMEM((IB, K), jnp.int32),
          pltpu.VMEM((NBUF, K, 128), F32),
          pltpu.VMEM((16, 128), F32),
          pltpu.VMEM_SHARED((npad, 128), F32),
      ] + [pltpu.SemaphoreType.DMA] * (2 * NBUF),
  )
  return fn(zs, src3, dst3)


def _dinv_blk(degp_ref, i):
  d = degp_ref[:, pl.ds(i * BLK, BLK)]
  return lax.rsqrt(d[0] + d[1] + 1.0)


def _mm_scale_body(x_ref, w_ref, degp_ref, o_ref):
  i = pl.program_id(0)
  dinv = _dinv_blk(degp_ref, i)
  z = jnp.dot(x_ref[...], w_ref[...], preferred_element_type=F32)
  o_ref[...] = z * dinv[:, None]


def _mid_body(agg_ref, zs_ref, degp_ref, b_ref, w_ref, o_ref):
  i = pl.program_id(0)
  dinv = _dinv_blk(degp_ref, i)[:, None]
  h = (agg_ref[0] + agg_ref[1] + zs_ref[...]) * dinv + b_ref[...]
  h = jnp.maximum(h, 0.0)
  o_ref[...] = jnp.dot(h, w_ref[...], preferred_element_type=F32) * dinv


def _pool_body(agg_ref, zs_ref, degp_ref, b_ref, batch_ref, wfc_ref, bfc_ref,
               o_ref, pacc, cacc, *, nblk, g):
  i = pl.program_id(0)
  dinv = _dinv_blk(degp_ref, i)[:, None]
  h = (agg_ref[0] + agg_ref[1] + zs_ref[...]) * dinv + b_ref[...]
  h = jnp.maximum(h, 0.0)
  bt = batch_ref[0, 0, :]
  mask = (bt[:, None] == lax.broadcasted_iota(jnp.int32, (BLK, g), 1))
  mask = mask.astype(F32)

  @pl.when(i == 0)
  def _():
    pacc[...] = jnp.zeros_like(pacc)
    cacc[...] = jnp.zeros_like(cacc)

  pacc[...] += lax.dot_general(mask, h, (((0,), (0,)), ((), ())),
                               preferred_element_type=F32)
  cacc[...] += jnp.sum(mask, axis=0)[None, :]

  @pl.when(i == nblk - 1)
  def _():
    cnt = jnp.maximum(cacc[...], 1.0).reshape(g, 1)
    pooled = pacc[...] / cnt
    o_ref[...] = (jnp.dot(pooled, wfc_ref[...], preferred_element_type=F32)
                  + bfc_ref[...])


def _tc_mm_scale(x_pad, w, degp, npad):
  nblk = npad // BLK
  return pl.pallas_call(
      _mm_scale_body,
      grid=(nblk,),
      in_specs=[
          pl.BlockSpec((BLK, 128), lambda i: (i, 0)),
          pl.BlockSpec((128, 128), lambda i: (0, 0)),
          pl.BlockSpec((NC, npad), lambda i: (0, 0)),
      ],
      out_specs=pl.BlockSpec((BLK, 128), lambda i: (i, 0)),
      out_shape=jax.ShapeDtypeStruct((npad, 128), F32),
  )(x_pad, w, degp)


def _tc_mid(agg, zs, degp, b, w, npad):
  nblk = npad // BLK
  return pl.pallas_call(
      _mid_body,
      grid=(nblk,),
      in_specs=[
          pl.BlockSpec((NC, BLK, 128), lambda i: (0, i, 0)),
          pl.BlockSpec((BLK, 128), lambda i: (i, 0)),
          pl.BlockSpec((NC, npad), lambda i: (0, 0)),
          pl.BlockSpec((1, 128), lambda i: (0, 0)),
          pl.BlockSpec((128, 128), lambda i: (0, 0)),
      ],
      out_specs=pl.BlockSpec((BLK, 128), lambda i: (i, 0)),
      out_shape=jax.ShapeDtypeStruct((npad, 128), F32),
  )(agg, zs, degp, b.reshape(1, 128), w)


def _tc_pool(agg, zs, degp, b, batch3, wfc, bfc, npad, g, c):
  nblk = npad // BLK
  return pl.pallas_call(
      functools.partial(_pool_body, nblk=nblk, g=g),
      grid=(nblk,),
      in_specs=[
          pl.BlockSpec((NC, BLK, 128), lambda i: (0, i, 0)),
          pl.BlockSpec((BLK, 128), lambda i: (i, 0)),
          pl.BlockSpec((NC, npad), lambda i: (0, 0)),
          pl.BlockSpec((1, 128), lambda i: (0, 0)),
          pl.BlockSpec((1, 1, BLK), lambda i: (i, 0, 0)),
          pl.BlockSpec((128, c), lambda i: (0, 0)),
          pl.BlockSpec((1, c), lambda i: (0, 0)),
      ],
      out_specs=pl.BlockSpec((g, c), lambda i: (0, 0)),
      out_shape=jax.ShapeDtypeStruct((g, c), F32),
      scratch_shapes=[
          pltpu.VMEM((g, 128), F32),
          pltpu.VMEM((1, g), F32),
      ],
  )(agg, zs, degp, b.reshape(1, 128), batch3, wfc, bfc.reshape(1, c))


def kernel(x, edge_index, batch, W1, b1, W2, b2, Wfc, bfc):
  n, d = x.shape
  e = edge_index.shape[1]
  g = 64
  c = Wfc.shape[1]

  npad = ((n + BLK) // BLK) * BLK
  ekc = NW * K * IB
  epad = -(-e // ekc) * ekc
  nch_agg = epad // (NW * K)
  nch_deg = epad // (NW * K)

  x_pad = jnp.pad(x, ((0, npad - n), (0, 0)))
  spread = n + jnp.arange(epad - e, dtype=jnp.int32) % (npad - n)
  src_pad = jnp.concatenate([edge_index[0], spread])
  dst_pad = jnp.concatenate([edge_index[1], spread])
  src3a = src_pad.reshape(NW, nch_agg, K)
  dst3a = dst_pad.reshape(NW, nch_agg, K)
  dst3d = dst_pad.reshape(NW, nch_deg, K)
  batch3 = jnp.concatenate(
      [batch, jnp.full((npad - n,), -1, jnp.int32)]).reshape(-1, 1, BLK)

  degp = _sc_deg(dst3d, npad)
  zs1 = _tc_mm_scale(x_pad, W1, degp, npad)
  agg1 = _sc_agg(zs1, src3a, dst3a, npad)
  zs2 = _tc_mid(agg1, zs1, degp, b1, W2, npad)
  agg2 = _sc_agg(zs2, src3a, dst3a, npad)
  return _tc_pool(agg2, zs2, degp, b2, batch3, Wfc, bfc, npad, g, c)
